# Initial kernel scaffold; baseline (speedup 1.0000x reference)
import jax
import jax.numpy as jnp
from jax.experimental import pallas as pl


def kernel(x, gate_w, gate_bias, W1, W2, W3, SW1, SW2, SW3):
    raise NotImplementedError("write your pallas kernel here")


# speedup vs baseline: 1.0223x; 1.0223x over previous
import functools

import jax
import jax.numpy as jnp
from jax import lax
from jax.experimental import pallas as pl
from jax.experimental.pallas import tpu as pltpu
from jax.experimental.pallas import tpu_sc as plsc

_T = 2048
_DIM = 1024
_INTER = 704
_E = 8
_BLK = 128
_RPAD = _T * 2 + _E * _BLK
_L = _RPAD + _T
_NB = _L // _BLK
_NBPAD = 64
_CHUNK = 128
_NCH = _T // _CHUNK
_NC = 2
_NS = 16
_NW = _NC * _NS
_TPW = _T // _NW


def _gate_body(x_ref, gw_ref, gb_ref, dest_ref, w0_ref, w1_ref, be_ref,
               bv_ref):
    x = x_ref[...]
    gw = gw_ref[...]
    logits = lax.dot_general(x, gw, (((1,), (1,)), ((), ())),
                             preferred_element_type=jnp.float32)
    scores = 1.0 / (1.0 + jnp.exp(-logits))
    biased = scores + gb_ref[...]
    e_iota = lax.broadcasted_iota(jnp.int32, (_T, _E), 1)

    m1 = jnp.max(biased, axis=1, keepdims=True)
    idx1 = jnp.min(jnp.where(biased >= m1, e_iota, _E), axis=1,
                   keepdims=True)
    oh0b = e_iota == idx1
    oh0 = oh0b.astype(jnp.float32)
    s1 = jnp.sum(oh0 * scores, axis=1, keepdims=True)
    biased2 = jnp.where(oh0b, -jnp.inf, biased)
    m2 = jnp.max(biased2, axis=1, keepdims=True)
    idx2 = jnp.min(jnp.where(biased2 >= m2, e_iota, _E), axis=1,
                   keepdims=True)
    oh1b = e_iota == idx2
    oh1 = oh1b.astype(jnp.float32)
    s2 = jnp.sum(oh1 * scores, axis=1, keepdims=True)
    denom = jnp.maximum(s1 + s2, 1e-10)
    w0_ref[...] = jnp.broadcast_to(s1 / denom, (_T, _E))
    w1_ref[...] = jnp.broadcast_to(s2 / denom, (_T, _E))

    r = lax.broadcasted_iota(jnp.int32, (_CHUNK, _CHUNK), 0)
    c = lax.broadcasted_iota(jnp.int32, (_CHUNK, _CHUNK), 1)
    tri = (r > c).astype(jnp.float32)
    r16 = lax.broadcasted_iota(jnp.int32, (_NCH, _NCH), 0)
    c16 = lax.broadcasted_iota(jnp.int32, (_NCH, _NCH), 1)
    tri16 = (r16 > c16).astype(jnp.float32)

    def excl_scan(oh):
        within, tots = [], []
        for k in range(_NCH):
            blk = oh[k * _CHUNK:(k + 1) * _CHUNK, :]
            within.append(jnp.dot(tri, blk, preferred_element_type=jnp.float32))
            tots.append(jnp.sum(blk, axis=0, keepdims=True))
        totals = jnp.concatenate(tots, axis=0)
        pref = jnp.dot(tri16, totals, preferred_element_type=jnp.float32)
        rows = [within[k] + pref[k:k + 1, :] for k in range(_NCH)]
        return jnp.concatenate(rows, axis=0), totals

    scan0, tot0_c = excl_scan(oh0)
    scan1, tot1_c = excl_scan(oh1)
    tot0 = jnp.sum(tot0_c, axis=0, keepdims=True)
    scan1 = scan1 + tot0
    cnt = tot0 + jnp.sum(tot1_c, axis=0, keepdims=True)

    cnt_i = cnt.astype(jnp.int32)
    pad_i = ((cnt_i + (_BLK - 1)) // _BLK) * _BLK
    pad = pad_i.astype(jnp.float32)
    r8 = lax.broadcasted_iota(jnp.int32, (_E, _E), 0)
    c8 = lax.broadcasted_iota(jnp.int32, (_E, _E), 1)
    up8 = (r8 < c8).astype(jnp.float32)
    offs = jnp.dot(pad, up8, preferred_element_type=jnp.float32)

    dest0 = jnp.sum(oh0 * (offs + scan0), axis=1)
    dest1 = jnp.sum(oh1 * (offs + scan1), axis=1)
    dest_ref[0, :] = dest0.astype(jnp.int32)
    dest_ref[1, :] = dest1.astype(jnp.int32)

    bstart = lax.broadcasted_iota(jnp.int32, (_NBPAD, 1), 0).astype(
        jnp.float32) * float(_BLK)
    offs9 = jnp.concatenate([offs, jnp.full((1, 1), float(_RPAD))], axis=1)
    cnt9 = jnp.concatenate([cnt, jnp.full((1, 1), float(_T))], axis=1)
    ge = (bstart >= offs9).astype(jnp.float32)
    blk_e = jnp.sum(ge, axis=1) - 1.0
    inseg = jnp.logical_and(bstart >= offs9, bstart < offs9 + cnt9)
    blk_v = jnp.sum(inseg.astype(jnp.float32), axis=1)
    be_ref[0, :] = blk_e.astype(jnp.int32)
    bv_ref[0, :] = blk_v.astype(jnp.int32)


_gate_call = pl.pallas_call(
    _gate_body,
    out_shape=[
        jax.ShapeDtypeStruct((2, _T), jnp.int32),
        jax.ShapeDtypeStruct((_T, _E), jnp.float32),
        jax.ShapeDtypeStruct((_T, _E), jnp.float32),
        jax.ShapeDtypeStruct((1, _NBPAD), jnp.int32),
        jax.ShapeDtypeStruct((1, _NBPAD), jnp.int32),
    ],
)

@functools.cache
def _sc_kernels():
    mesh = plsc.VectorSubcoreMesh(core_axis_name="c", subcore_axis_name="s",
                                  num_cores=_NC, num_subcores=_NS)

    @functools.partial(
        pl.kernel,
        out_type=jax.ShapeDtypeStruct((_L, _DIM), jnp.float32),
        mesh=mesh,
        scratch_types=[
            pltpu.VMEM((_TPW, _DIM), jnp.float32),
            pltpu.VMEM((_TPW,), jnp.int32),
            pltpu.VMEM((_TPW,), jnp.int32),
            pltpu.SemaphoreType.DMA,
            pltpu.SemaphoreType.DMA,
        ],
    )
    def dispatch(x_hbm, d0_hbm, d1_hbm, xg_hbm, xbuf, i0, i1, s0, s1):
        wid = lax.axis_index("s") * _NC + lax.axis_index("c")
        tb = wid * _TPW
        pltpu.sync_copy(x_hbm.at[pl.ds(tb, _TPW)], xbuf)
        pltpu.sync_copy(d0_hbm.at[pl.ds(tb, _TPW)], i0)
        pltpu.sync_copy(d1_hbm.at[pl.ds(tb, _TPW)], i1)
        c0 = pltpu.async_copy(xbuf, xg_hbm.at[i0], s0)
        c1 = pltpu.async_copy(xbuf, xg_hbm.at[i1], s1)
        pltpu.sync_copy(xbuf, xg_hbm.at[pl.ds(_RPAD + tb, _TPW)])
        c0.wait()
        c1.wait()

    @functools.partial(
        pl.kernel,
        out_type=(jax.ShapeDtypeStruct((_T, _DIM), jnp.float32),
                  jax.ShapeDtypeStruct((_T, _DIM), jnp.float32)),
        mesh=mesh,
        scratch_types=[
            pltpu.VMEM((_TPW, _DIM), jnp.float32),
            pltpu.VMEM((_TPW,), jnp.int32),
            pltpu.SemaphoreType.DMA,
        ],
    )
    def gather_back(out_hbm, d0_hbm, d1_hbm, z0_hbm, z1_hbm, buf, idx, sem):
        wid = lax.axis_index("s") * _NC + lax.axis_index("c")
        tb = wid * _TPW
        pltpu.sync_copy(d0_hbm.at[pl.ds(tb, _TPW)], idx)
        pltpu.async_copy(out_hbm.at[idx], buf, sem).wait()
        pltpu.sync_copy(buf, z0_hbm.at[pl.ds(tb, _TPW)])
        pltpu.sync_copy(d1_hbm.at[pl.ds(tb, _TPW)], idx)
        pltpu.async_copy(out_hbm.at[idx], buf, sem).wait()
        pltpu.sync_copy(buf, z1_hbm.at[pl.ds(tb, _TPW)])

    return dispatch, gather_back


def _ffn_body(be_ref, bv_ref, xg_ref, w1_ref, w2_ref, w3_ref, o_ref):
    b = pl.program_id(0)

    @pl.when(bv_ref[b] != 0)
    def _():
        xb = xg_ref[...]
        w1 = w1_ref[0]
        w2 = w2_ref[0]
        w3 = w3_ref[0]
        h1 = lax.dot_general(xb, w1, (((1,), (1,)), ((), ())),
                             preferred_element_type=jnp.float32)
        h3 = lax.dot_general(xb, w3, (((1,), (1,)), ((), ())),
                             preferred_element_type=jnp.float32)
        h = h1 / (1.0 + jnp.exp(-h1)) * h3
        o_ref[...] = lax.dot_general(h, w2, (((1,), (1,)), ((), ())),
                                     preferred_element_type=jnp.float32)


_ffn_grid = pltpu.PrefetchScalarGridSpec(
    num_scalar_prefetch=2,
    grid=(_NB,),
    in_specs=[
        pl.BlockSpec((_BLK, _DIM), lambda b, be, bv: (b, 0)),
        pl.BlockSpec((1, _INTER, _DIM), lambda b, be, bv: (be[b], 0, 0)),
        pl.BlockSpec((1, _DIM, _INTER), lambda b, be, bv: (be[b], 0, 0)),
        pl.BlockSpec((1, _INTER, _DIM), lambda b, be, bv: (be[b], 0, 0)),
    ],
    out_specs=pl.BlockSpec((_BLK, _DIM), lambda b, be, bv: (b, 0)),
)

_ffn_call = pl.pallas_call(
    _ffn_body,
    grid_spec=_ffn_grid,
    out_shape=jax.ShapeDtypeStruct((_L, _DIM), jnp.float32),
)


_CB = 256


def _combine_body(z0_ref, z1_ref, os_ref, w0_ref, w1_ref, y_ref):
    w0 = w0_ref[:, 0:1]
    w1 = w1_ref[:, 0:1]
    y_ref[...] = w0 * z0_ref[...] + w1 * z1_ref[...] + os_ref[...]


_combine_call = pl.pallas_call(
    _combine_body,
    grid=(_T // _CB,),
    in_specs=[
        pl.BlockSpec((_CB, _DIM), lambda i: (i, 0)),
        pl.BlockSpec((_CB, _DIM), lambda i: (i, 0)),
        pl.BlockSpec((_CB, _DIM), lambda i: (_RPAD // _CB + i, 0)),
        pl.BlockSpec((_CB, _E), lambda i: (i, 0)),
        pl.BlockSpec((_CB, _E), lambda i: (i, 0)),
    ],
    out_specs=pl.BlockSpec((_CB, _DIM), lambda i: (i, 0)),
    out_shape=jax.ShapeDtypeStruct((_T, _DIM), jnp.float32),
)


def kernel(x, gate_w, gate_bias, W1, W2, W3, SW1, SW2, SW3):
    gb = gate_bias.reshape(1, _E).astype(jnp.float32)
    dispatch, gather_back = _sc_kernels()
    dest, w0b, w1b, be, bv = _gate_call(x, gate_w, gb)
    d0 = dest[0]
    d1 = dest[1]
    xg = dispatch(x, d0, d1)
    w1c = jnp.concatenate([W1, SW1], axis=0)
    w2c = jnp.concatenate([W2, SW2], axis=0)
    w3c = jnp.concatenate([W3, SW3], axis=0)
    be_ = be.reshape(_NBPAD)[:_NB]
    bv_ = bv.reshape(_NBPAD)[:_NB]
    out = _ffn_call(be_, bv_, xg, w1c, w2c, w3c)
    z0, z1 = gather_back(out, d0, d1)
    return _combine_call(z0, z1, out, w0b, w1b)


# speedup vs baseline: 1.2532x; 1.2259x over previous
import functools

import jax
import jax.numpy as jnp
from jax import lax
from jax.experimental import pallas as pl
from jax.experimental.pallas import tpu as pltpu
from jax.experimental.pallas import tpu_sc as plsc

_T = 2048
_DIM = 1024
_INTER = 704
_E = 8
_BLK = 128
_RPAD = _T * 2 + _E * _BLK
_L = _RPAD + _T
_NB = _RPAD // _BLK
_NBPAD = 64
_CHUNK = 128
_NCH = _T // _CHUNK
_NC = 2
_NS = 16
_NW = _NC * _NS
_TPW = _T // _NW


def _gate_body(x_ref, gw_ref, gb_ref, dest_ref, w0_ref, w1_ref, be_ref,
               bv_ref):
    x = x_ref[...]
    gw = gw_ref[...]
    logits = lax.dot_general(x, gw, (((1,), (1,)), ((), ())),
                             preferred_element_type=jnp.float32)
    scores = 1.0 / (1.0 + jnp.exp(-logits))
    biased = scores + gb_ref[...]
    e_iota = lax.broadcasted_iota(jnp.int32, (_T, _E), 1)

    m1 = jnp.max(biased, axis=1, keepdims=True)
    idx1 = jnp.min(jnp.where(biased >= m1, e_iota, _E), axis=1,
                   keepdims=True)
    oh0b = e_iota == idx1
    oh0 = oh0b.astype(jnp.float32)
    s1 = jnp.sum(oh0 * scores, axis=1, keepdims=True)
    biased2 = jnp.where(oh0b, -jnp.inf, biased)
    m2 = jnp.max(biased2, axis=1, keepdims=True)
    idx2 = jnp.min(jnp.where(biased2 >= m2, e_iota, _E), axis=1,
                   keepdims=True)
    oh1b = e_iota == idx2
    oh1 = oh1b.astype(jnp.float32)
    s2 = jnp.sum(oh1 * scores, axis=1, keepdims=True)
    denom = jnp.maximum(s1 + s2, 1e-10)
    w0_ref[...] = jnp.broadcast_to(s1 / denom, (_T, _E))
    w1_ref[...] = jnp.broadcast_to(s2 / denom, (_T, _E))

    r = lax.broadcasted_iota(jnp.int32, (_CHUNK, _CHUNK), 0)
    c = lax.broadcasted_iota(jnp.int32, (_CHUNK, _CHUNK), 1)
    tri = (r > c).astype(jnp.float32)
    r16 = lax.broadcasted_iota(jnp.int32, (_NCH, _NCH), 0)
    c16 = lax.broadcasted_iota(jnp.int32, (_NCH, _NCH), 1)
    tri16 = (r16 > c16).astype(jnp.float32)

    def excl_scan(oh):
        within, tots = [], []
        for k in range(_NCH):
            blk = oh[k * _CHUNK:(k + 1) * _CHUNK, :]
            within.append(jnp.dot(tri, blk, preferred_element_type=jnp.float32))
            tots.append(jnp.sum(blk, axis=0, keepdims=True))
        totals = jnp.concatenate(tots, axis=0)
        pref = jnp.dot(tri16, totals, preferred_element_type=jnp.float32)
        rows = [within[k] + pref[k:k + 1, :] for k in range(_NCH)]
        return jnp.concatenate(rows, axis=0), totals

    scan0, tot0_c = excl_scan(oh0)
    scan1, tot1_c = excl_scan(oh1)
    tot0 = jnp.sum(tot0_c, axis=0, keepdims=True)
    scan1 = scan1 + tot0
    cnt = tot0 + jnp.sum(tot1_c, axis=0, keepdims=True)

    cnt_i = cnt.astype(jnp.int32)
    pad_i = ((cnt_i + (_BLK - 1)) // _BLK) * _BLK
    pad = pad_i.astype(jnp.float32)
    r8 = lax.broadcasted_iota(jnp.int32, (_E, _E), 0)
    c8 = lax.broadcasted_iota(jnp.int32, (_E, _E), 1)
    up8 = (r8 < c8).astype(jnp.float32)
    offs = jnp.dot(pad, up8, preferred_element_type=jnp.float32)

    dest0 = jnp.sum(oh0 * (offs + scan0), axis=1)
    dest1 = jnp.sum(oh1 * (offs + scan1), axis=1)
    dest_ref[0, :] = dest0.astype(jnp.int32)
    dest_ref[1, :] = dest1.astype(jnp.int32)

    bstart = lax.broadcasted_iota(jnp.int32, (_NBPAD, 1), 0).astype(
        jnp.float32) * float(_BLK)
    offs9 = jnp.concatenate([offs, jnp.full((1, 1), float(_RPAD))], axis=1)
    cnt9 = jnp.concatenate([cnt, jnp.full((1, 1), float(_T))], axis=1)
    ge = (bstart >= offs9).astype(jnp.float32)
    blk_e = jnp.sum(ge, axis=1) - 1.0
    inseg = jnp.logical_and(bstart >= offs9, bstart < offs9 + cnt9)
    blk_v = jnp.sum(inseg.astype(jnp.float32), axis=1)
    be_ref[0, :] = blk_e.astype(jnp.int32)
    bv_ref[0, :] = blk_v.astype(jnp.int32)


_gate_call = pl.pallas_call(
    _gate_body,
    out_shape=[
        jax.ShapeDtypeStruct((2, _T), jnp.int32),
        jax.ShapeDtypeStruct((_T, _E), jnp.float32),
        jax.ShapeDtypeStruct((_T, _E), jnp.float32),
        jax.ShapeDtypeStruct((1, _NBPAD), jnp.int32),
        jax.ShapeDtypeStruct((1, _NBPAD), jnp.int32),
    ],
)

@functools.cache
def _sc_kernels():
    mesh = plsc.VectorSubcoreMesh(core_axis_name="c", subcore_axis_name="s",
                                  num_cores=_NC, num_subcores=_NS)

    @functools.partial(
        pl.kernel,
        out_type=jax.ShapeDtypeStruct((_RPAD, _DIM), jnp.float32),
        mesh=mesh,
        scratch_types=[
            pltpu.VMEM((_TPW, _DIM), jnp.float32),
            pltpu.VMEM((_TPW,), jnp.int32),
            pltpu.VMEM((_TPW,), jnp.int32),
            pltpu.SemaphoreType.DMA,
            pltpu.SemaphoreType.DMA,
        ],
    )
    def dispatch(x_hbm, d0_hbm, d1_hbm, xg_hbm, xbuf, i0, i1, s0, s1):
        wid = lax.axis_index("s") * _NC + lax.axis_index("c")
        tb = wid * _TPW
        pltpu.sync_copy(x_hbm.at[pl.ds(tb, _TPW)], xbuf)
        pltpu.sync_copy(d0_hbm.at[pl.ds(tb, _TPW)], i0)
        pltpu.sync_copy(d1_hbm.at[pl.ds(tb, _TPW)], i1)
        c0 = pltpu.async_copy(xbuf, xg_hbm.at[i0], s0)
        c1 = pltpu.async_copy(xbuf, xg_hbm.at[i1], s1)
        c0.wait()
        c1.wait()

    @functools.partial(
        pl.kernel,
        out_type=(jax.ShapeDtypeStruct((_T, _DIM), jnp.float32),
                  jax.ShapeDtypeStruct((_T, _DIM), jnp.float32)),
        mesh=mesh,
        scratch_types=[
            pltpu.VMEM((_TPW, _DIM), jnp.float32),
            pltpu.VMEM((_TPW,), jnp.int32),
            pltpu.SemaphoreType.DMA,
        ],
    )
    def gather_back(out_hbm, d0_hbm, d1_hbm, z0_hbm, z1_hbm, buf, idx, sem):
        wid = lax.axis_index("s") * _NC + lax.axis_index("c")
        tb = wid * _TPW
        pltpu.sync_copy(d0_hbm.at[pl.ds(tb, _TPW)], idx)
        pltpu.async_copy(out_hbm.at[idx], buf, sem).wait()
        pltpu.sync_copy(buf, z0_hbm.at[pl.ds(tb, _TPW)])
        pltpu.sync_copy(d1_hbm.at[pl.ds(tb, _TPW)], idx)
        pltpu.async_copy(out_hbm.at[idx], buf, sem).wait()
        pltpu.sync_copy(buf, z1_hbm.at[pl.ds(tb, _TPW)])

    return dispatch, gather_back


def _swiglu(xb, w1, w2, w3):
    xb = xb.astype(jnp.bfloat16)
    h1 = lax.dot_general(xb, w1.astype(jnp.bfloat16), (((1,), (1,)), ((), ())),
                         preferred_element_type=jnp.float32)
    h3 = lax.dot_general(xb, w3.astype(jnp.bfloat16), (((1,), (1,)), ((), ())),
                         preferred_element_type=jnp.float32)
    h = (h1 / (1.0 + jnp.exp(-h1)) * h3).astype(jnp.bfloat16)
    return lax.dot_general(h, w2.astype(jnp.bfloat16), (((1,), (1,)), ((), ())),
                           preferred_element_type=jnp.float32)


def _ffn_body(be_ref, bv_ref, xg_ref, w1_ref, w2_ref, w3_ref, o_ref):
    b = pl.program_id(0)

    @pl.when(bv_ref[b] != 0)
    def _():
        o_ref[...] = _swiglu(xg_ref[...], w1_ref[0], w2_ref[0], w3_ref[0])


_ffn_grid = pltpu.PrefetchScalarGridSpec(
    num_scalar_prefetch=2,
    grid=(_NB,),
    in_specs=[
        pl.BlockSpec((_BLK, _DIM), lambda b, be, bv: (b, 0)),
        pl.BlockSpec((1, _INTER, _DIM), lambda b, be, bv: (be[b], 0, 0)),
        pl.BlockSpec((1, _DIM, _INTER), lambda b, be, bv: (be[b], 0, 0)),
        pl.BlockSpec((1, _INTER, _DIM), lambda b, be, bv: (be[b], 0, 0)),
    ],
    out_specs=pl.BlockSpec((_BLK, _DIM), lambda b, be, bv: (b, 0)),
)

_ffn_call = pl.pallas_call(
    _ffn_body,
    grid_spec=_ffn_grid,
    out_shape=jax.ShapeDtypeStruct((_RPAD, _DIM), jnp.float32),
)


def _sffn_body(x_ref, w1_ref, w2_ref, w3_ref, o_ref):
    o_ref[...] = _swiglu(x_ref[...], w1_ref[0], w2_ref[0], w3_ref[0])


_sffn_call = pl.pallas_call(
    _sffn_body,
    grid=(_T // _BLK,),
    in_specs=[
        pl.BlockSpec((_BLK, _DIM), lambda i: (i, 0)),
        pl.BlockSpec((1, _INTER, _DIM), lambda i: (0, 0, 0)),
        pl.BlockSpec((1, _DIM, _INTER), lambda i: (0, 0, 0)),
        pl.BlockSpec((1, _INTER, _DIM), lambda i: (0, 0, 0)),
    ],
    out_specs=pl.BlockSpec((_BLK, _DIM), lambda i: (i, 0)),
    out_shape=jax.ShapeDtypeStruct((_T, _DIM), jnp.float32),
)


_CB = 256


def _combine_body(z0_ref, z1_ref, os_ref, w0_ref, w1_ref, y_ref):
    w0 = w0_ref[:, 0:1]
    w1 = w1_ref[:, 0:1]
    y_ref[...] = w0 * z0_ref[...] + w1 * z1_ref[...] + os_ref[...]


_combine_call = pl.pallas_call(
    _combine_body,
    grid=(_T // _CB,),
    in_specs=[
        pl.BlockSpec((_CB, _DIM), lambda i: (i, 0)),
        pl.BlockSpec((_CB, _DIM), lambda i: (i, 0)),
        pl.BlockSpec((_CB, _DIM), lambda i: (i, 0)),
        pl.BlockSpec((_CB, _E), lambda i: (i, 0)),
        pl.BlockSpec((_CB, _E), lambda i: (i, 0)),
    ],
    out_specs=pl.BlockSpec((_CB, _DIM), lambda i: (i, 0)),
    out_shape=jax.ShapeDtypeStruct((_T, _DIM), jnp.float32),
)


def kernel(x, gate_w, gate_bias, W1, W2, W3, SW1, SW2, SW3):
    gb = gate_bias.reshape(1, _E).astype(jnp.float32)
    dispatch, gather_back = _sc_kernels()
    dest, w0b, w1b, be, bv = _gate_call(x, gate_w, gb)
    d0 = dest[0]
    d1 = dest[1]
    xg = dispatch(x, d0, d1)
    be_ = be.reshape(_NBPAD)[:_NB]
    bv_ = bv.reshape(_NBPAD)[:_NB]
    out = _ffn_call(be_, bv_, xg, W1, W2, W3)
    out_s = _sffn_call(x, SW1, SW2, SW3)
    z0, z1 = gather_back(out, d0, d1)
    return _combine_call(z0, z1, out_s, w0b, w1b)


# speedup vs baseline: 1.4919x; 1.1904x over previous
import functools

import jax
import jax.numpy as jnp
from jax import lax
from jax.experimental import pallas as pl
from jax.experimental.pallas import tpu as pltpu
from jax.experimental.pallas import tpu_sc as plsc

_T = 2048
_DIM = 1024
_INTER = 704
_E = 8
_BLK = 128
_RPAD = _T * 2 + _E * _BLK
_L = _RPAD + _T
_NB = _RPAD // _BLK
_NBPAD = 64
_CHUNK = 128
_NCH = _T // _CHUNK
_NC = 2
_NS = 16
_NW = _NC * _NS
_TPW = _T // _NW


def _gate_body(x_ref, gw_ref, gb_ref, dest_ref, w0_ref, w1_ref, be_ref,
               bv_ref):
    x = x_ref[...]
    gw = gw_ref[...]
    logits = lax.dot_general(x, gw, (((1,), (1,)), ((), ())),
                             preferred_element_type=jnp.float32)
    scores = 1.0 / (1.0 + jnp.exp(-logits))
    biased = scores + gb_ref[...]
    e_iota = lax.broadcasted_iota(jnp.int32, (_T, _E), 1)

    m1 = jnp.max(biased, axis=1, keepdims=True)
    idx1 = jnp.min(jnp.where(biased >= m1, e_iota, _E), axis=1,
                   keepdims=True)
    oh0b = e_iota == idx1
    oh0 = oh0b.astype(jnp.float32)
    s1 = jnp.sum(oh0 * scores, axis=1, keepdims=True)
    biased2 = jnp.where(oh0b, -jnp.inf, biased)
    m2 = jnp.max(biased2, axis=1, keepdims=True)
    idx2 = jnp.min(jnp.where(biased2 >= m2, e_iota, _E), axis=1,
                   keepdims=True)
    oh1b = e_iota == idx2
    oh1 = oh1b.astype(jnp.float32)
    s2 = jnp.sum(oh1 * scores, axis=1, keepdims=True)
    denom = jnp.maximum(s1 + s2, 1e-10)
    w0_ref[...] = jnp.broadcast_to(s1 / denom, (_T, _E))
    w1_ref[...] = jnp.broadcast_to(s2 / denom, (_T, _E))

    r = lax.broadcasted_iota(jnp.int32, (_CHUNK, _CHUNK), 0)
    c = lax.broadcasted_iota(jnp.int32, (_CHUNK, _CHUNK), 1)
    tri = (r > c).astype(jnp.float32)
    r16 = lax.broadcasted_iota(jnp.int32, (_NCH, _NCH), 0)
    c16 = lax.broadcasted_iota(jnp.int32, (_NCH, _NCH), 1)
    tri16 = (r16 > c16).astype(jnp.float32)

    def excl_scan(oh):
        within, tots = [], []
        for k in range(_NCH):
            blk = oh[k * _CHUNK:(k + 1) * _CHUNK, :]
            within.append(jnp.dot(tri, blk, preferred_element_type=jnp.float32))
            tots.append(jnp.sum(blk, axis=0, keepdims=True))
        totals = jnp.concatenate(tots, axis=0)
        pref = jnp.dot(tri16, totals, preferred_element_type=jnp.float32)
        rows = [within[k] + pref[k:k + 1, :] for k in range(_NCH)]
        return jnp.concatenate(rows, axis=0), totals

    scan0, tot0_c = excl_scan(oh0)
    scan1, tot1_c = excl_scan(oh1)
    tot0 = jnp.sum(tot0_c, axis=0, keepdims=True)
    scan1 = scan1 + tot0
    cnt = tot0 + jnp.sum(tot1_c, axis=0, keepdims=True)

    cnt_i = cnt.astype(jnp.int32)
    pad_i = ((cnt_i + (_BLK - 1)) // _BLK) * _BLK
    pad = pad_i.astype(jnp.float32)
    r8 = lax.broadcasted_iota(jnp.int32, (_E, _E), 0)
    c8 = lax.broadcasted_iota(jnp.int32, (_E, _E), 1)
    up8 = (r8 < c8).astype(jnp.float32)
    offs = jnp.dot(pad, up8, preferred_element_type=jnp.float32)

    dest0 = jnp.sum(oh0 * (offs + scan0), axis=1)
    dest1 = jnp.sum(oh1 * (offs + scan1), axis=1)
    dest_ref[0, :] = dest0.astype(jnp.int32)
    dest_ref[1, :] = dest1.astype(jnp.int32)

    bstart = lax.broadcasted_iota(jnp.int32, (_NBPAD, 1), 0).astype(
        jnp.float32) * float(_BLK)
    offs9 = jnp.concatenate([offs, jnp.full((1, 1), float(_RPAD))], axis=1)
    cnt9 = jnp.concatenate([cnt, jnp.full((1, 1), float(_T))], axis=1)
    ge = (bstart >= offs9).astype(jnp.float32)
    blk_e = jnp.sum(ge, axis=1) - 1.0
    inseg = jnp.logical_and(bstart >= offs9, bstart < offs9 + cnt9)
    blk_v = jnp.sum(inseg.astype(jnp.float32), axis=1)
    be_ref[0, :] = blk_e.astype(jnp.int32)
    bv_ref[0, :] = blk_v.astype(jnp.int32)


_gate_call = pl.pallas_call(
    _gate_body,
    out_shape=[
        jax.ShapeDtypeStruct((2, _T), jnp.int32),
        jax.ShapeDtypeStruct((_T, _E), jnp.float32),
        jax.ShapeDtypeStruct((_T, _E), jnp.float32),
        jax.ShapeDtypeStruct((1, _NBPAD), jnp.int32),
        jax.ShapeDtypeStruct((1, _NBPAD), jnp.int32),
    ],
)

@functools.cache
def _sc_kernels():
    mesh = plsc.VectorSubcoreMesh(core_axis_name="c", subcore_axis_name="s",
                                  num_cores=_NC, num_subcores=_NS)

    @functools.partial(
        pl.kernel,
        out_type=jax.ShapeDtypeStruct((_RPAD, _DIM), jnp.float32),
        mesh=mesh,
        scratch_types=[
            pltpu.VMEM((_TPW, _DIM), jnp.float32),
            pltpu.VMEM((_TPW,), jnp.int32),
            pltpu.VMEM((_TPW,), jnp.int32),
            pltpu.SemaphoreType.DMA,
            pltpu.SemaphoreType.DMA,
        ],
    )
    def dispatch(x_hbm, d0_hbm, d1_hbm, xg_hbm, xbuf, i0, i1, s0, s1):
        wid = lax.axis_index("s") * _NC + lax.axis_index("c")
        tb = wid * _TPW
        pltpu.sync_copy(x_hbm.at[pl.ds(tb, _TPW)], xbuf)
        pltpu.sync_copy(d0_hbm.at[pl.ds(tb, _TPW)], i0)
        pltpu.sync_copy(d1_hbm.at[pl.ds(tb, _TPW)], i1)
        c0 = pltpu.async_copy(xbuf, xg_hbm.at[i0], s0)
        c1 = pltpu.async_copy(xbuf, xg_hbm.at[i1], s1)
        c0.wait()
        c1.wait()

    @functools.partial(
        pl.kernel,
        out_type=(jax.ShapeDtypeStruct((_T, _DIM), jnp.float32),
                  jax.ShapeDtypeStruct((_T, _DIM), jnp.float32)),
        mesh=mesh,
        scratch_types=[
            pltpu.VMEM((_TPW, _DIM), jnp.float32),
            pltpu.VMEM((_TPW,), jnp.int32),
            pltpu.SemaphoreType.DMA,
        ],
    )
    def gather_back(out_hbm, d0_hbm, d1_hbm, z0_hbm, z1_hbm, buf, idx, sem):
        wid = lax.axis_index("s") * _NC + lax.axis_index("c")
        tb = wid * _TPW
        pltpu.sync_copy(d0_hbm.at[pl.ds(tb, _TPW)], idx)
        pltpu.async_copy(out_hbm.at[idx], buf, sem).wait()
        pltpu.sync_copy(buf, z0_hbm.at[pl.ds(tb, _TPW)])
        pltpu.sync_copy(d1_hbm.at[pl.ds(tb, _TPW)], idx)
        pltpu.async_copy(out_hbm.at[idx], buf, sem).wait()
        pltpu.sync_copy(buf, z1_hbm.at[pl.ds(tb, _TPW)])

    return dispatch, gather_back


def _swiglu_cached(xb, w1s, w2s, w3s):
    xb = xb.astype(jnp.bfloat16)
    h1 = lax.dot_general(xb, w1s[...], (((1,), (1,)), ((), ())),
                         preferred_element_type=jnp.float32)
    h3 = lax.dot_general(xb, w3s[...], (((1,), (1,)), ((), ())),
                         preferred_element_type=jnp.float32)
    h = (h1 / (1.0 + jnp.exp(-h1)) * h3).astype(jnp.bfloat16)
    return lax.dot_general(h, w2s[...], (((1,), (0,)), ((), ())),
                           preferred_element_type=jnp.float32)


def _ffn_body(be_ref, bv_ref, xg_ref, w1_ref, w2_ref, w3_ref, o_ref,
              w1s, w2s, w3s):
    b = pl.program_id(0)
    bm1 = jnp.maximum(b - 1, 0)
    changed = jnp.logical_or(b == 0, be_ref[b] != be_ref[bm1])

    @pl.when(jnp.logical_and(changed, bv_ref[b] != 0))
    def _():
        w1s[...] = w1_ref[0].astype(jnp.bfloat16)
        w2s[...] = w2_ref[0].astype(jnp.bfloat16)
        w3s[...] = w3_ref[0].astype(jnp.bfloat16)

    @pl.when(bv_ref[b] != 0)
    def _():
        o_ref[...] = _swiglu_cached(xg_ref[...], w1s, w2s, w3s)


_ffn_grid = pltpu.PrefetchScalarGridSpec(
    num_scalar_prefetch=2,
    grid=(_NB,),
    in_specs=[
        pl.BlockSpec((_BLK, _DIM), lambda b, be, bv: (b, 0)),
        pl.BlockSpec((1, _INTER, _DIM), lambda b, be, bv: (be[b], 0, 0)),
        pl.BlockSpec((1, _INTER, _DIM), lambda b, be, bv: (be[b], 0, 0)),
        pl.BlockSpec((1, _INTER, _DIM), lambda b, be, bv: (be[b], 0, 0)),
    ],
    out_specs=pl.BlockSpec((_BLK, _DIM), lambda b, be, bv: (b, 0)),
    scratch_shapes=[
        pltpu.VMEM((_INTER, _DIM), jnp.bfloat16),
        pltpu.VMEM((_INTER, _DIM), jnp.bfloat16),
        pltpu.VMEM((_INTER, _DIM), jnp.bfloat16),
    ],
)

_ffn_call = pl.pallas_call(
    _ffn_body,
    grid_spec=_ffn_grid,
    out_shape=jax.ShapeDtypeStruct((_RPAD, _DIM), jnp.float32),
)


def _sffn_body(x_ref, w1_ref, w2_ref, w3_ref, o_ref, w1s, w2s, w3s):
    @pl.when(pl.program_id(0) == 0)
    def _():
        w1s[...] = w1_ref[0].astype(jnp.bfloat16)
        w2s[...] = w2_ref[0].astype(jnp.bfloat16)
        w3s[...] = w3_ref[0].astype(jnp.bfloat16)

    o_ref[...] = _swiglu_cached(x_ref[...], w1s, w2s, w3s)


_sffn_call = pl.pallas_call(
    _sffn_body,
    grid=(_T // _BLK,),
    in_specs=[
        pl.BlockSpec((_BLK, _DIM), lambda i: (i, 0)),
        pl.BlockSpec((1, _INTER, _DIM), lambda i: (0, 0, 0)),
        pl.BlockSpec((1, _INTER, _DIM), lambda i: (0, 0, 0)),
        pl.BlockSpec((1, _INTER, _DIM), lambda i: (0, 0, 0)),
    ],
    out_specs=pl.BlockSpec((_BLK, _DIM), lambda i: (i, 0)),
    out_shape=jax.ShapeDtypeStruct((_T, _DIM), jnp.float32),
    scratch_shapes=[
        pltpu.VMEM((_INTER, _DIM), jnp.bfloat16),
        pltpu.VMEM((_INTER, _DIM), jnp.bfloat16),
        pltpu.VMEM((_INTER, _DIM), jnp.bfloat16),
    ],
)


_CB = 256


def _combine_body(z0_ref, z1_ref, os_ref, w0_ref, w1_ref, y_ref):
    w0 = w0_ref[:, 0:1]
    w1 = w1_ref[:, 0:1]
    y_ref[...] = w0 * z0_ref[...] + w1 * z1_ref[...] + os_ref[...]


_combine_call = pl.pallas_call(
    _combine_body,
    grid=(_T // _CB,),
    in_specs=[
        pl.BlockSpec((_CB, _DIM), lambda i: (i, 0)),
        pl.BlockSpec((_CB, _DIM), lambda i: (i, 0)),
        pl.BlockSpec((_CB, _DIM), lambda i: (i, 0)),
        pl.BlockSpec((_CB, _E), lambda i: (i, 0)),
        pl.BlockSpec((_CB, _E), lambda i: (i, 0)),
    ],
    out_specs=pl.BlockSpec((_CB, _DIM), lambda i: (i, 0)),
    out_shape=jax.ShapeDtypeStruct((_T, _DIM), jnp.float32),
)


def kernel(x, gate_w, gate_bias, W1, W2, W3, SW1, SW2, SW3):
    gb = gate_bias.reshape(1, _E).astype(jnp.float32)
    dispatch, gather_back = _sc_kernels()
    dest, w0b, w1b, be, bv = _gate_call(x, gate_w, gb)
    d0 = dest[0]
    d1 = dest[1]
    xg = dispatch(x, d0, d1)
    be_ = be.reshape(_NBPAD)[:_NB]
    bv_ = bv.reshape(_NBPAD)[:_NB]
    out = _ffn_call(be_, bv_, xg, W1, jnp.swapaxes(W2, 1, 2), W3)
    out_s = _sffn_call(x, SW1, jnp.swapaxes(SW2, 1, 2), SW3)
    z0, z1 = gather_back(out, d0, d1)
    return _combine_call(z0, z1, out_s, w0b, w1b)


# speedup vs baseline: 1.8043x; 1.2094x over previous
import functools

import jax
import jax.numpy as jnp
from jax import lax
from jax.experimental import pallas as pl
from jax.experimental.pallas import tpu as pltpu
from jax.experimental.pallas import tpu_sc as plsc

_T = 2048
_DIM = 1024
_INTER = 704
_E = 8
_BLK = 256
_RPAD = _T * 2 + _E * _BLK
_L = _RPAD + _T
_NB = _RPAD // _BLK
_NBPAD = 64
_CHUNK = 128
_NCH = _T // _CHUNK
_NC = 2
_NS = 16
_NW = _NC * _NS
_TPW = _T // _NW


def _gate_body(x_ref, gw_ref, gb_ref, dest_ref, w0_ref, w1_ref, be_ref,
               bv_ref):
    x = x_ref[...]
    gw = gw_ref[...]
    logits = lax.dot_general(x, gw, (((1,), (1,)), ((), ())),
                             preferred_element_type=jnp.float32)
    scores = 1.0 / (1.0 + jnp.exp(-logits))
    biased = scores + gb_ref[...]
    e_iota = lax.broadcasted_iota(jnp.int32, (_T, _E), 1)

    m1 = jnp.max(biased, axis=1, keepdims=True)
    idx1 = jnp.min(jnp.where(biased >= m1, e_iota, _E), axis=1,
                   keepdims=True)
    oh0b = e_iota == idx1
    oh0 = oh0b.astype(jnp.float32)
    s1 = jnp.sum(oh0 * scores, axis=1, keepdims=True)
    biased2 = jnp.where(oh0b, -jnp.inf, biased)
    m2 = jnp.max(biased2, axis=1, keepdims=True)
    idx2 = jnp.min(jnp.where(biased2 >= m2, e_iota, _E), axis=1,
                   keepdims=True)
    oh1b = e_iota == idx2
    oh1 = oh1b.astype(jnp.float32)
    s2 = jnp.sum(oh1 * scores, axis=1, keepdims=True)
    denom = jnp.maximum(s1 + s2, 1e-10)
    w0_ref[...] = jnp.broadcast_to(s1 / denom, (_T, _E))
    w1_ref[...] = jnp.broadcast_to(s2 / denom, (_T, _E))

    r = lax.broadcasted_iota(jnp.int32, (_CHUNK, _CHUNK), 0)
    c = lax.broadcasted_iota(jnp.int32, (_CHUNK, _CHUNK), 1)
    tri = (r > c).astype(jnp.float32)
    r16 = lax.broadcasted_iota(jnp.int32, (_NCH, _NCH), 0)
    c16 = lax.broadcasted_iota(jnp.int32, (_NCH, _NCH), 1)
    tri16 = (r16 > c16).astype(jnp.float32)

    def excl_scan(oh):
        within, tots = [], []
        for k in range(_NCH):
            blk = oh[k * _CHUNK:(k + 1) * _CHUNK, :]
            within.append(jnp.dot(tri, blk, preferred_element_type=jnp.float32))
            tots.append(jnp.sum(blk, axis=0, keepdims=True))
        totals = jnp.concatenate(tots, axis=0)
        pref = jnp.dot(tri16, totals, preferred_element_type=jnp.float32)
        rows = [within[k] + pref[k:k + 1, :] for k in range(_NCH)]
        return jnp.concatenate(rows, axis=0), totals

    scan0, tot0_c = excl_scan(oh0)
    scan1, tot1_c = excl_scan(oh1)
    tot0 = jnp.sum(tot0_c, axis=0, keepdims=True)
    scan1 = scan1 + tot0
    cnt = tot0 + jnp.sum(tot1_c, axis=0, keepdims=True)

    cnt_i = cnt.astype(jnp.int32)
    pad_i = ((cnt_i + (_BLK - 1)) // _BLK) * _BLK
    pad = pad_i.astype(jnp.float32)
    r8 = lax.broadcasted_iota(jnp.int32, (_E, _E), 0)
    c8 = lax.broadcasted_iota(jnp.int32, (_E, _E), 1)
    up8 = (r8 < c8).astype(jnp.float32)
    offs = jnp.dot(pad, up8, preferred_element_type=jnp.float32)

    dest0 = jnp.sum(oh0 * (offs + scan0), axis=1)
    dest1 = jnp.sum(oh1 * (offs + scan1), axis=1)
    dest_ref[0, :] = dest0.astype(jnp.int32)
    dest_ref[1, :] = dest1.astype(jnp.int32)

    bstart = lax.broadcasted_iota(jnp.int32, (_NBPAD, 1), 0).astype(
        jnp.float32) * float(_BLK)
    offs9 = jnp.concatenate([offs, jnp.full((1, 1), float(_RPAD))], axis=1)
    cnt9 = jnp.concatenate([cnt, jnp.full((1, 1), float(_T))], axis=1)
    ge = (bstart >= offs9).astype(jnp.float32)
    blk_e = jnp.sum(ge, axis=1) - 1.0
    inseg = jnp.logical_and(bstart >= offs9, bstart < offs9 + cnt9)
    blk_v = jnp.sum(inseg.astype(jnp.float32), axis=1)
    be_ref[0, :] = blk_e.astype(jnp.int32)
    bv_ref[0, :] = blk_v.astype(jnp.int32)


_gate_call = pl.pallas_call(
    _gate_body,
    out_shape=[
        jax.ShapeDtypeStruct((2, _T), jnp.int32),
        jax.ShapeDtypeStruct((_T, _E), jnp.float32),
        jax.ShapeDtypeStruct((_T, _E), jnp.float32),
        jax.ShapeDtypeStruct((1, _NBPAD), jnp.int32),
        jax.ShapeDtypeStruct((1, _NBPAD), jnp.int32),
    ],
)

@functools.cache
def _sc_kernels():
    mesh = plsc.VectorSubcoreMesh(core_axis_name="c", subcore_axis_name="s",
                                  num_cores=_NC, num_subcores=_NS)

    @functools.partial(
        pl.kernel,
        out_type=jax.ShapeDtypeStruct((_RPAD, _DIM), jnp.float32),
        mesh=mesh,
        scratch_types=[
            pltpu.VMEM((_TPW, _DIM), jnp.float32),
            pltpu.VMEM((_TPW,), jnp.int32),
            pltpu.VMEM((_TPW,), jnp.int32),
            pltpu.SemaphoreType.DMA,
            pltpu.SemaphoreType.DMA,
        ],
    )
    def dispatch(x_hbm, d0_hbm, d1_hbm, xg_hbm, xbuf, i0, i1, s0, s1):
        wid = lax.axis_index("s") * _NC + lax.axis_index("c")
        tb = wid * _TPW
        pltpu.sync_copy(x_hbm.at[pl.ds(tb, _TPW)], xbuf)
        pltpu.sync_copy(d0_hbm.at[pl.ds(tb, _TPW)], i0)
        pltpu.sync_copy(d1_hbm.at[pl.ds(tb, _TPW)], i1)
        c0 = pltpu.async_copy(xbuf, xg_hbm.at[i0], s0)
        c1 = pltpu.async_copy(xbuf, xg_hbm.at[i1], s1)
        c0.wait()
        c1.wait()

    @functools.partial(
        pl.kernel,
        out_type=(jax.ShapeDtypeStruct((_T, _DIM), jnp.float32),
                  jax.ShapeDtypeStruct((_T, _DIM), jnp.float32)),
        mesh=mesh,
        scratch_types=[
            pltpu.VMEM((_TPW, _DIM), jnp.float32),
            pltpu.VMEM((_TPW,), jnp.int32),
            pltpu.SemaphoreType.DMA,
        ],
    )
    def gather_back(out_hbm, d0_hbm, d1_hbm, z0_hbm, z1_hbm, buf, idx, sem):
        wid = lax.axis_index("s") * _NC + lax.axis_index("c")
        tb = wid * _TPW
        pltpu.sync_copy(d0_hbm.at[pl.ds(tb, _TPW)], idx)
        pltpu.async_copy(out_hbm.at[idx], buf, sem).wait()
        pltpu.sync_copy(buf, z0_hbm.at[pl.ds(tb, _TPW)])
        pltpu.sync_copy(d1_hbm.at[pl.ds(tb, _TPW)], idx)
        pltpu.async_copy(out_hbm.at[idx], buf, sem).wait()
        pltpu.sync_copy(buf, z1_hbm.at[pl.ds(tb, _TPW)])

    return dispatch, gather_back


def _swiglu_cached(xb, w1s, w2s, w3s):
    xb = xb.astype(jnp.bfloat16)
    h1 = lax.dot_general(xb, w1s[...], (((1,), (1,)), ((), ())),
                         preferred_element_type=jnp.float32)
    h3 = lax.dot_general(xb, w3s[...], (((1,), (1,)), ((), ())),
                         preferred_element_type=jnp.float32)
    h = (h1 / (1.0 + jnp.exp(-h1)) * h3).astype(jnp.bfloat16)
    return lax.dot_general(h, w2s[...], (((1,), (0,)), ((), ())),
                           preferred_element_type=jnp.float32)


def _ffn_body(be_ref, bv_ref, xg_ref, w1_ref, w2_ref, w3_ref, o_ref,
              w1s, w2s, w3s):
    b = pl.program_id(0)
    bm1 = jnp.maximum(b - 1, 0)
    changed = jnp.logical_or(b == 0, be_ref[b] != be_ref[bm1])

    @pl.when(jnp.logical_and(changed, bv_ref[b] != 0))
    def _():
        w1s[...] = w1_ref[0].astype(jnp.bfloat16)
        w2s[...] = w2_ref[0].astype(jnp.bfloat16)
        w3s[...] = w3_ref[0].astype(jnp.bfloat16)

    @pl.when(bv_ref[b] != 0)
    def _():
        o_ref[...] = _swiglu_cached(xg_ref[...], w1s, w2s, w3s)


_ffn_grid = pltpu.PrefetchScalarGridSpec(
    num_scalar_prefetch=2,
    grid=(_NB,),
    in_specs=[
        pl.BlockSpec((_BLK, _DIM), lambda b, be, bv: (b, 0)),
        pl.BlockSpec((1, _INTER, _DIM), lambda b, be, bv: (be[b], 0, 0)),
        pl.BlockSpec((1, _INTER, _DIM), lambda b, be, bv: (be[b], 0, 0)),
        pl.BlockSpec((1, _INTER, _DIM), lambda b, be, bv: (be[b], 0, 0)),
    ],
    out_specs=pl.BlockSpec((_BLK, _DIM), lambda b, be, bv: (b, 0)),
    scratch_shapes=[
        pltpu.VMEM((_INTER, _DIM), jnp.bfloat16),
        pltpu.VMEM((_INTER, _DIM), jnp.bfloat16),
        pltpu.VMEM((_INTER, _DIM), jnp.bfloat16),
    ],
)

_ffn_call = pl.pallas_call(
    _ffn_body,
    grid_spec=_ffn_grid,
    out_shape=jax.ShapeDtypeStruct((_RPAD, _DIM), jnp.float32),
)


def _sffn_body(x_ref, w1_ref, w2_ref, w3_ref, o_ref, w1s, w2s, w3s):
    @pl.when(pl.program_id(0) == 0)
    def _():
        w1s[...] = w1_ref[0].astype(jnp.bfloat16)
        w2s[...] = w2_ref[0].astype(jnp.bfloat16)
        w3s[...] = w3_ref[0].astype(jnp.bfloat16)

    o_ref[...] = _swiglu_cached(x_ref[...], w1s, w2s, w3s)


_sffn_call = pl.pallas_call(
    _sffn_body,
    grid=(_T // _BLK,),
    in_specs=[
        pl.BlockSpec((_BLK, _DIM), lambda i: (i, 0)),
        pl.BlockSpec((1, _INTER, _DIM), lambda i: (0, 0, 0)),
        pl.BlockSpec((1, _INTER, _DIM), lambda i: (0, 0, 0)),
        pl.BlockSpec((1, _INTER, _DIM), lambda i: (0, 0, 0)),
    ],
    out_specs=pl.BlockSpec((_BLK, _DIM), lambda i: (i, 0)),
    out_shape=jax.ShapeDtypeStruct((_T, _DIM), jnp.float32),
    scratch_shapes=[
        pltpu.VMEM((_INTER, _DIM), jnp.bfloat16),
        pltpu.VMEM((_INTER, _DIM), jnp.bfloat16),
        pltpu.VMEM((_INTER, _DIM), jnp.bfloat16),
    ],
)


_CB = 256


def _combine_body(z0_ref, z1_ref, os_ref, w0_ref, w1_ref, y_ref):
    w0 = w0_ref[:, 0:1]
    w1 = w1_ref[:, 0:1]
    y_ref[...] = w0 * z0_ref[...] + w1 * z1_ref[...] + os_ref[...]


_combine_call = pl.pallas_call(
    _combine_body,
    grid=(_T // _CB,),
    in_specs=[
        pl.BlockSpec((_CB, _DIM), lambda i: (i, 0)),
        pl.BlockSpec((_CB, _DIM), lambda i: (i, 0)),
        pl.BlockSpec((_CB, _DIM), lambda i: (i, 0)),
        pl.BlockSpec((_CB, _E), lambda i: (i, 0)),
        pl.BlockSpec((_CB, _E), lambda i: (i, 0)),
    ],
    out_specs=pl.BlockSpec((_CB, _DIM), lambda i: (i, 0)),
    out_shape=jax.ShapeDtypeStruct((_T, _DIM), jnp.float32),
)


def kernel(x, gate_w, gate_bias, W1, W2, W3, SW1, SW2, SW3):
    gb = gate_bias.reshape(1, _E).astype(jnp.float32)
    dispatch, gather_back = _sc_kernels()
    dest, w0b, w1b, be, bv = _gate_call(x, gate_w, gb)
    d0 = dest[0]
    d1 = dest[1]
    xg = dispatch(x, d0, d1)
    be_ = be.reshape(_NBPAD)[:_NB]
    bv_ = bv.reshape(_NBPAD)[:_NB]
    out = _ffn_call(be_, bv_, xg, W1, jnp.swapaxes(W2, 1, 2), W3)
    z0, z1 = gather_back(out, d0, d1)
    out_s = _sffn_call(x, SW1, jnp.swapaxes(SW2, 1, 2), SW3)
    return _combine_call(z0, z1, out_s, w0b, w1b)


# speedup vs baseline: 1.9691x; 1.0913x over previous
import functools

import jax
import jax.numpy as jnp
from jax import lax
from jax.experimental import pallas as pl
from jax.experimental.pallas import tpu as pltpu
from jax.experimental.pallas import tpu_sc as plsc

_T = 2048
_DIM = 1024
_INTER = 704
_E = 8
_BLK = 256
_RPAD = _T * 2 + _E * _BLK
_L = _RPAD + _T
_NB = _RPAD // _BLK
_NBPAD = 64
_CHUNK = 128
_NCH = _T // _CHUNK
_NC = 2
_NS = 16
_NW = _NC * _NS
_TPW = _T // _NW


def _gate_body(x_ref, gw_ref, gb_ref, dest_ref, w0_ref, w1_ref, be_ref,
               bv_ref, nx_ref):
    x = x_ref[...]
    gw = gw_ref[...]
    logits = lax.dot_general(x, gw, (((1,), (1,)), ((), ())),
                             preferred_element_type=jnp.float32)
    scores = 1.0 / (1.0 + jnp.exp(-logits))
    biased = scores + gb_ref[...]
    e_iota = lax.broadcasted_iota(jnp.int32, (_T, _E), 1)

    m1 = jnp.max(biased, axis=1, keepdims=True)
    idx1 = jnp.min(jnp.where(biased >= m1, e_iota, _E), axis=1,
                   keepdims=True)
    oh0b = e_iota == idx1
    oh0 = oh0b.astype(jnp.float32)
    s1 = jnp.sum(oh0 * scores, axis=1, keepdims=True)
    biased2 = jnp.where(oh0b, -jnp.inf, biased)
    m2 = jnp.max(biased2, axis=1, keepdims=True)
    idx2 = jnp.min(jnp.where(biased2 >= m2, e_iota, _E), axis=1,
                   keepdims=True)
    oh1b = e_iota == idx2
    oh1 = oh1b.astype(jnp.float32)
    s2 = jnp.sum(oh1 * scores, axis=1, keepdims=True)
    denom = jnp.maximum(s1 + s2, 1e-10)
    w0_ref[...] = jnp.broadcast_to(s1 / denom, (_T, _E))
    w1_ref[...] = jnp.broadcast_to(s2 / denom, (_T, _E))

    r = lax.broadcasted_iota(jnp.int32, (_CHUNK, _CHUNK), 0)
    c = lax.broadcasted_iota(jnp.int32, (_CHUNK, _CHUNK), 1)
    tri = (r > c).astype(jnp.float32)
    r16 = lax.broadcasted_iota(jnp.int32, (_NCH, _NCH), 0)
    c16 = lax.broadcasted_iota(jnp.int32, (_NCH, _NCH), 1)
    tri16 = (r16 > c16).astype(jnp.float32)

    def excl_scan(oh):
        within, tots = [], []
        for k in range(_NCH):
            blk = oh[k * _CHUNK:(k + 1) * _CHUNK, :]
            within.append(jnp.dot(tri, blk, preferred_element_type=jnp.float32))
            tots.append(jnp.sum(blk, axis=0, keepdims=True))
        totals = jnp.concatenate(tots, axis=0)
        pref = jnp.dot(tri16, totals, preferred_element_type=jnp.float32)
        rows = [within[k] + pref[k:k + 1, :] for k in range(_NCH)]
        return jnp.concatenate(rows, axis=0), totals

    scan0, tot0_c = excl_scan(oh0)
    scan1, tot1_c = excl_scan(oh1)
    tot0 = jnp.sum(tot0_c, axis=0, keepdims=True)
    scan1 = scan1 + tot0
    cnt = tot0 + jnp.sum(tot1_c, axis=0, keepdims=True)

    cnt_i = cnt.astype(jnp.int32)
    pad_i = ((cnt_i + (_BLK - 1)) // _BLK) * _BLK
    pad = pad_i.astype(jnp.float32)
    r8 = lax.broadcasted_iota(jnp.int32, (_E, _E), 0)
    c8 = lax.broadcasted_iota(jnp.int32, (_E, _E), 1)
    up8 = (r8 < c8).astype(jnp.float32)
    offs = jnp.dot(pad, up8, preferred_element_type=jnp.float32)

    dest0 = jnp.sum(oh0 * (offs + scan0), axis=1)
    dest1 = jnp.sum(oh1 * (offs + scan1), axis=1)
    dest_ref[0, :] = dest0.astype(jnp.int32)
    dest_ref[1, :] = dest1.astype(jnp.int32)

    bstart = lax.broadcasted_iota(jnp.int32, (_NBPAD, 1), 0).astype(
        jnp.float32) * float(_BLK)
    offs9 = jnp.concatenate([offs, jnp.full((1, 1), float(_RPAD))], axis=1)
    cnt9 = jnp.concatenate([cnt, jnp.full((1, 1), float(_T))], axis=1)
    ge = (bstart >= offs9).astype(jnp.float32)
    blk_e = jnp.sum(ge, axis=1) - 1.0
    inseg = jnp.logical_and(bstart >= offs9, bstart < offs9 + cnt9)
    blk_v = jnp.sum(inseg.astype(jnp.float32), axis=1)
    be_ref[0, :] = blk_e.astype(jnp.int32)
    bv_ref[0, :] = blk_v.astype(jnp.int32)

    eidx = lax.broadcasted_iota(jnp.int32, (_NBPAD, _E), 1)
    blk_ei = blk_e.astype(jnp.int32)
    cand = jnp.logical_and(cnt > 0.0, eidx > blk_ei[:, None])
    nxt = jnp.min(jnp.where(cand, eidx, 99), axis=1)
    nx_ref[0, :] = jnp.where(nxt > 98, blk_ei, nxt)


_gate_call = pl.pallas_call(
    _gate_body,
    out_shape=[
        jax.ShapeDtypeStruct((2, _T), jnp.int32),
        jax.ShapeDtypeStruct((_T, _E), jnp.float32),
        jax.ShapeDtypeStruct((_T, _E), jnp.float32),
        jax.ShapeDtypeStruct((1, _NBPAD), jnp.int32),
        jax.ShapeDtypeStruct((1, _NBPAD), jnp.int32),
        jax.ShapeDtypeStruct((1, _NBPAD), jnp.int32),
    ],
)

@functools.cache
def _sc_kernels():
    mesh = plsc.VectorSubcoreMesh(core_axis_name="c", subcore_axis_name="s",
                                  num_cores=_NC, num_subcores=_NS)

    @functools.partial(
        pl.kernel,
        out_type=jax.ShapeDtypeStruct((_RPAD, _DIM), jnp.float32),
        mesh=mesh,
        scratch_types=[
            pltpu.VMEM((_TPW, _DIM), jnp.float32),
            pltpu.VMEM((_TPW,), jnp.int32),
            pltpu.VMEM((_TPW,), jnp.int32),
            pltpu.SemaphoreType.DMA,
            pltpu.SemaphoreType.DMA,
        ],
    )
    def dispatch(x_hbm, d0_hbm, d1_hbm, xg_hbm, xbuf, i0, i1, s0, s1):
        wid = lax.axis_index("s") * _NC + lax.axis_index("c")
        tb = wid * _TPW
        pltpu.sync_copy(x_hbm.at[pl.ds(tb, _TPW)], xbuf)
        pltpu.sync_copy(d0_hbm.at[pl.ds(tb, _TPW)], i0)
        pltpu.sync_copy(d1_hbm.at[pl.ds(tb, _TPW)], i1)
        c0 = pltpu.async_copy(xbuf, xg_hbm.at[i0], s0)
        c1 = pltpu.async_copy(xbuf, xg_hbm.at[i1], s1)
        c0.wait()
        c1.wait()

    @functools.partial(
        pl.kernel,
        out_type=(jax.ShapeDtypeStruct((_T, _DIM), jnp.float32),
                  jax.ShapeDtypeStruct((_T, _DIM), jnp.float32)),
        mesh=mesh,
        scratch_types=[
            pltpu.VMEM((_TPW, _DIM), jnp.float32),
            pltpu.VMEM((_TPW,), jnp.int32),
            pltpu.SemaphoreType.DMA,
        ],
    )
    def gather_back(out_hbm, d0_hbm, d1_hbm, z0_hbm, z1_hbm, buf, idx, sem):
        wid = lax.axis_index("s") * _NC + lax.axis_index("c")
        tb = wid * _TPW
        pltpu.sync_copy(d0_hbm.at[pl.ds(tb, _TPW)], idx)
        pltpu.async_copy(out_hbm.at[idx], buf, sem).wait()
        pltpu.sync_copy(buf, z0_hbm.at[pl.ds(tb, _TPW)])
        pltpu.sync_copy(d1_hbm.at[pl.ds(tb, _TPW)], idx)
        pltpu.async_copy(out_hbm.at[idx], buf, sem).wait()
        pltpu.sync_copy(buf, z1_hbm.at[pl.ds(tb, _TPW)])

    return dispatch, gather_back


def _swiglu_cached(xb, w1s, w2s, w3s):
    xb = xb.astype(jnp.bfloat16)
    h1 = lax.dot_general(xb, w1s[...], (((1,), (1,)), ((), ())),
                         preferred_element_type=jnp.float32)
    h3 = lax.dot_general(xb, w3s[...], (((1,), (1,)), ((), ())),
                         preferred_element_type=jnp.float32)
    h = (h1 / (1.0 + jnp.exp(-h1)) * h3).astype(jnp.bfloat16)
    return lax.dot_general(h, w2s[...], (((1,), (0,)), ((), ())),
                           preferred_element_type=jnp.float32)


def _ffn_body(be_ref, bv_ref, nx_ref, xg_ref, w1_hbm, w2_hbm, w3_hbm, o_ref,
              wfa, wfb, cache, sema, semb):
    b = pl.program_id(0)
    e = be_ref[b]
    bm1 = jnp.maximum(b - 1, 0)
    changed = jnp.logical_or(b == 0, e != be_ref[bm1])
    nxt = nx_ref[b]
    slot = e % 2
    nslot = nxt % 2

    def issue(ee, wf, sem):
        pltpu.async_copy(w1_hbm.at[ee], wf.at[0], sem.at[0])
        pltpu.async_copy(w2_hbm.at[ee], wf.at[1], sem.at[1])
        pltpu.async_copy(w3_hbm.at[ee], wf.at[2], sem.at[2])

    def drain(ee, wf, sem):
        pltpu.make_async_copy(w1_hbm.at[ee], wf.at[0], sem.at[0]).wait()
        pltpu.make_async_copy(w2_hbm.at[ee], wf.at[1], sem.at[1]).wait()
        pltpu.make_async_copy(w3_hbm.at[ee], wf.at[2], sem.at[2]).wait()

    for s, wf, sem in ((0, wfa, sema), (1, wfb, semb)):
        @pl.when(jnp.logical_and(b == 0, slot == s))
        def _(wf=wf, sem=sem):
            issue(e, wf, sem)

    @pl.when(changed)
    def _():
        for s, wf, sem in ((0, wfa, sema), (1, wfb, semb)):
            @pl.when(slot == s)
            def _(wf=wf, sem=sem):
                drain(e, wf, sem)
                cache[0, :, :] = wf[0].astype(jnp.bfloat16)
                cache[1, :, :] = wf[1].astype(jnp.bfloat16)
                cache[2, :, :] = wf[2].astype(jnp.bfloat16)

        for s, wf, sem in ((0, wfa, sema), (1, wfb, semb)):
            @pl.when(nslot == s)
            def _(wf=wf, sem=sem):
                issue(nxt, wf, sem)

    @pl.when(bv_ref[b] != 0)
    def _():
        xb = xg_ref[...].astype(jnp.bfloat16)
        h1 = lax.dot_general(xb, cache[0], (((1,), (1,)), ((), ())),
                             preferred_element_type=jnp.float32)
        h3 = lax.dot_general(xb, cache[2], (((1,), (1,)), ((), ())),
                             preferred_element_type=jnp.float32)
        h = (h1 / (1.0 + jnp.exp(-h1)) * h3).astype(jnp.bfloat16)
        o_ref[...] = lax.dot_general(h, cache[1], (((1,), (0,)), ((), ())),
                                     preferred_element_type=jnp.float32)

    for s, wf, sem in ((0, wfa, sema), (1, wfb, semb)):
        @pl.when(jnp.logical_and(b == _NB - 1, nslot == s))
        def _(wf=wf, sem=sem):
            drain(nxt, wf, sem)


_ffn_grid = pltpu.PrefetchScalarGridSpec(
    num_scalar_prefetch=3,
    grid=(_NB,),
    in_specs=[
        pl.BlockSpec((_BLK, _DIM), lambda b, be, bv, nx: (b, 0)),
        pl.BlockSpec(memory_space=pltpu.MemorySpace.HBM),
        pl.BlockSpec(memory_space=pltpu.MemorySpace.HBM),
        pl.BlockSpec(memory_space=pltpu.MemorySpace.HBM),
    ],
    out_specs=pl.BlockSpec((_BLK, _DIM), lambda b, be, bv, nx: (b, 0)),
    scratch_shapes=[
        pltpu.VMEM((3, _INTER, _DIM), jnp.float32),
        pltpu.VMEM((3, _INTER, _DIM), jnp.float32),
        pltpu.VMEM((3, _INTER, _DIM), jnp.bfloat16),
        pltpu.SemaphoreType.DMA((3,)),
        pltpu.SemaphoreType.DMA((3,)),
    ],
)

_ffn_call = pl.pallas_call(
    _ffn_body,
    grid_spec=_ffn_grid,
    out_shape=jax.ShapeDtypeStruct((_RPAD, _DIM), jnp.float32),
)


def _sffn_body(x_ref, w1_ref, w2_ref, w3_ref, o_ref, w1s, w2s, w3s):
    @pl.when(pl.program_id(0) == 0)
    def _():
        w1s[...] = w1_ref[0].astype(jnp.bfloat16)
        w2s[...] = w2_ref[0].astype(jnp.bfloat16)
        w3s[...] = w3_ref[0].astype(jnp.bfloat16)

    o_ref[...] = _swiglu_cached(x_ref[...], w1s, w2s, w3s)


_sffn_call = pl.pallas_call(
    _sffn_body,
    grid=(_T // _BLK,),
    in_specs=[
        pl.BlockSpec((_BLK, _DIM), lambda i: (i, 0)),
        pl.BlockSpec((1, _INTER, _DIM), lambda i: (0, 0, 0)),
        pl.BlockSpec((1, _INTER, _DIM), lambda i: (0, 0, 0)),
        pl.BlockSpec((1, _INTER, _DIM), lambda i: (0, 0, 0)),
    ],
    out_specs=pl.BlockSpec((_BLK, _DIM), lambda i: (i, 0)),
    out_shape=jax.ShapeDtypeStruct((_T, _DIM), jnp.float32),
    scratch_shapes=[
        pltpu.VMEM((_INTER, _DIM), jnp.bfloat16),
        pltpu.VMEM((_INTER, _DIM), jnp.bfloat16),
        pltpu.VMEM((_INTER, _DIM), jnp.bfloat16),
    ],
)


_CB = 256


def _combine_body(z0_ref, z1_ref, os_ref, w0_ref, w1_ref, y_ref):
    w0 = w0_ref[:, 0:1]
    w1 = w1_ref[:, 0:1]
    y_ref[...] = w0 * z0_ref[...] + w1 * z1_ref[...] + os_ref[...]


_combine_call = pl.pallas_call(
    _combine_body,
    grid=(_T // _CB,),
    in_specs=[
        pl.BlockSpec((_CB, _DIM), lambda i: (i, 0)),
        pl.BlockSpec((_CB, _DIM), lambda i: (i, 0)),
        pl.BlockSpec((_CB, _DIM), lambda i: (i, 0)),
        pl.BlockSpec((_CB, _E), lambda i: (i, 0)),
        pl.BlockSpec((_CB, _E), lambda i: (i, 0)),
    ],
    out_specs=pl.BlockSpec((_CB, _DIM), lambda i: (i, 0)),
    out_shape=jax.ShapeDtypeStruct((_T, _DIM), jnp.float32),
)


def kernel(x, gate_w, gate_bias, W1, W2, W3, SW1, SW2, SW3):
    gb = gate_bias.reshape(1, _E).astype(jnp.float32)
    dispatch, gather_back = _sc_kernels()
    dest, w0b, w1b, be, bv, nx = _gate_call(x, gate_w, gb)
    d0 = dest[0]
    d1 = dest[1]
    xg = dispatch(x, d0, d1)
    be_ = be.reshape(_NBPAD)[:_NB]
    bv_ = bv.reshape(_NBPAD)[:_NB]
    nx_ = nx.reshape(_NBPAD)[:_NB]
    out = _ffn_call(be_, bv_, nx_, xg, W1, jnp.swapaxes(W2, 1, 2), W3)
    z0, z1 = gather_back(out, d0, d1)
    out_s = _sffn_call(x, SW1, jnp.swapaxes(SW2, 1, 2), SW3)
    return _combine_call(z0, z1, out_s, w0b, w1b)


# speedup vs baseline: 1.9816x; 1.0063x over previous
import functools

import jax
import jax.numpy as jnp
from jax import lax
from jax.experimental import pallas as pl
from jax.experimental.pallas import tpu as pltpu
from jax.experimental.pallas import tpu_sc as plsc

_T = 2048
_DIM = 1024
_INTER = 704
_E = 8
_BLK = 256
_RPAD = _T * 2 + _E * _BLK
_L = _RPAD + _T
_NB = _RPAD // _BLK
_NBPAD = 64
_CHUNK = 256
_NCH = _T // _CHUNK
_NC = 2
_NS = 16
_NW = _NC * _NS
_TPW = _T // _NW


def _gate_body(x_ref, gw_ref, gb_ref, dest_ref, w0_ref, w1_ref, be_ref,
               bv_ref, nx_ref):
    x = x_ref[...]
    gw = gw_ref[...]
    logits = lax.dot_general(x, gw, (((1,), (1,)), ((), ())),
                             preferred_element_type=jnp.float32)
    scores = 1.0 / (1.0 + jnp.exp(-logits))
    biased = scores + gb_ref[...]
    e_iota = lax.broadcasted_iota(jnp.int32, (_T, _E), 1)

    m1 = jnp.max(biased, axis=1, keepdims=True)
    idx1 = jnp.min(jnp.where(biased >= m1, e_iota, _E), axis=1,
                   keepdims=True)
    oh0b = e_iota == idx1
    oh0 = oh0b.astype(jnp.float32)
    s1 = jnp.sum(oh0 * scores, axis=1, keepdims=True)
    biased2 = jnp.where(oh0b, -jnp.inf, biased)
    m2 = jnp.max(biased2, axis=1, keepdims=True)
    idx2 = jnp.min(jnp.where(biased2 >= m2, e_iota, _E), axis=1,
                   keepdims=True)
    oh1b = e_iota == idx2
    oh1 = oh1b.astype(jnp.float32)
    s2 = jnp.sum(oh1 * scores, axis=1, keepdims=True)
    denom = jnp.maximum(s1 + s2, 1e-10)
    w0_ref[...] = jnp.broadcast_to(s1 / denom, (_T, _E))
    w1_ref[...] = jnp.broadcast_to(s2 / denom, (_T, _E))

    r = lax.broadcasted_iota(jnp.int32, (_CHUNK, _CHUNK), 0)
    c = lax.broadcasted_iota(jnp.int32, (_CHUNK, _CHUNK), 1)
    tri = (r > c).astype(jnp.float32)
    r16 = lax.broadcasted_iota(jnp.int32, (_NCH, _NCH), 0)
    c16 = lax.broadcasted_iota(jnp.int32, (_NCH, _NCH), 1)
    tri16 = (r16 > c16).astype(jnp.float32)

    def excl_scan(oh):
        within, tots = [], []
        for k in range(_NCH):
            blk = oh[k * _CHUNK:(k + 1) * _CHUNK, :]
            within.append(jnp.dot(tri, blk, preferred_element_type=jnp.float32))
            tots.append(jnp.sum(blk, axis=0, keepdims=True))
        totals = jnp.concatenate(tots, axis=0)
        pref = jnp.dot(tri16, totals, preferred_element_type=jnp.float32)
        rows = [within[k] + pref[k:k + 1, :] for k in range(_NCH)]
        return jnp.concatenate(rows, axis=0), totals

    scan0, tot0_c = excl_scan(oh0)
    scan1, tot1_c = excl_scan(oh1)
    tot0 = jnp.sum(tot0_c, axis=0, keepdims=True)
    scan1 = scan1 + tot0
    cnt = tot0 + jnp.sum(tot1_c, axis=0, keepdims=True)

    cnt_i = cnt.astype(jnp.int32)
    pad_i = ((cnt_i + (_BLK - 1)) // _BLK) * _BLK
    pad = pad_i.astype(jnp.float32)
    r8 = lax.broadcasted_iota(jnp.int32, (_E, _E), 0)
    c8 = lax.broadcasted_iota(jnp.int32, (_E, _E), 1)
    up8 = (r8 < c8).astype(jnp.float32)
    offs = jnp.dot(pad, up8, preferred_element_type=jnp.float32)

    dest0 = jnp.sum(oh0 * (offs + scan0), axis=1)
    dest1 = jnp.sum(oh1 * (offs + scan1), axis=1)
    dest_ref[0, :] = dest0.astype(jnp.int32)
    dest_ref[1, :] = dest1.astype(jnp.int32)

    bstart = lax.broadcasted_iota(jnp.int32, (_NBPAD, 1), 0).astype(
        jnp.float32) * float(_BLK)
    offs9 = jnp.concatenate([offs, jnp.full((1, 1), float(_RPAD))], axis=1)
    cnt9 = jnp.concatenate([cnt, jnp.full((1, 1), float(_T))], axis=1)
    ge = (bstart >= offs9).astype(jnp.float32)
    blk_e = jnp.sum(ge, axis=1) - 1.0
    inseg = jnp.logical_and(bstart >= offs9, bstart < offs9 + cnt9)
    blk_v = jnp.sum(inseg.astype(jnp.float32), axis=1)
    be_ref[0, :] = blk_e.astype(jnp.int32)
    bv_ref[0, :] = blk_v.astype(jnp.int32)

    eidx = lax.broadcasted_iota(jnp.int32, (_NBPAD, _E), 1)
    blk_ei = blk_e.astype(jnp.int32)
    cand = jnp.logical_and(cnt > 0.0, eidx > blk_ei[:, None])
    nxt = jnp.min(jnp.where(cand, eidx, 99), axis=1)
    nx_ref[0, :] = jnp.where(nxt > 98, blk_ei, nxt)


_gate_call = pl.pallas_call(
    _gate_body,
    out_shape=[
        jax.ShapeDtypeStruct((2, _T), jnp.int32),
        jax.ShapeDtypeStruct((_T, _E), jnp.float32),
        jax.ShapeDtypeStruct((_T, _E), jnp.float32),
        jax.ShapeDtypeStruct((1, _NBPAD), jnp.int32),
        jax.ShapeDtypeStruct((1, _NBPAD), jnp.int32),
        jax.ShapeDtypeStruct((1, _NBPAD), jnp.int32),
    ],
)

@functools.cache
def _sc_kernels():
    mesh = plsc.VectorSubcoreMesh(core_axis_name="c", subcore_axis_name="s",
                                  num_cores=_NC, num_subcores=_NS)

    @functools.partial(
        pl.kernel,
        out_type=jax.ShapeDtypeStruct((_RPAD, _DIM), jnp.float32),
        mesh=mesh,
        scratch_types=[
            pltpu.VMEM((_TPW, _DIM), jnp.float32),
            pltpu.VMEM((_TPW,), jnp.int32),
            pltpu.VMEM((_TPW,), jnp.int32),
            pltpu.SemaphoreType.DMA,
            pltpu.SemaphoreType.DMA,
        ],
    )
    def dispatch(x_hbm, d0_hbm, d1_hbm, xg_hbm, xbuf, i0, i1, s0, s1):
        wid = lax.axis_index("s") * _NC + lax.axis_index("c")
        tb = wid * _TPW
        pltpu.sync_copy(x_hbm.at[pl.ds(tb, _TPW)], xbuf)
        pltpu.sync_copy(d0_hbm.at[pl.ds(tb, _TPW)], i0)
        pltpu.sync_copy(d1_hbm.at[pl.ds(tb, _TPW)], i1)
        c0 = pltpu.async_copy(xbuf, xg_hbm.at[i0], s0)
        c1 = pltpu.async_copy(xbuf, xg_hbm.at[i1], s1)
        c0.wait()
        c1.wait()

    @functools.partial(
        pl.kernel,
        out_type=(jax.ShapeDtypeStruct((_T, _DIM), jnp.float32),
                  jax.ShapeDtypeStruct((_T, _DIM), jnp.float32)),
        mesh=mesh,
        scratch_types=[
            pltpu.VMEM((_TPW, _DIM), jnp.float32),
            pltpu.VMEM((_TPW,), jnp.int32),
            pltpu.SemaphoreType.DMA,
        ],
    )
    def gather_back(out_hbm, d0_hbm, d1_hbm, z0_hbm, z1_hbm, buf, idx, sem):
        wid = lax.axis_index("s") * _NC + lax.axis_index("c")
        tb = wid * _TPW
        pltpu.sync_copy(d0_hbm.at[pl.ds(tb, _TPW)], idx)
        pltpu.async_copy(out_hbm.at[idx], buf, sem).wait()
        pltpu.sync_copy(buf, z0_hbm.at[pl.ds(tb, _TPW)])
        pltpu.sync_copy(d1_hbm.at[pl.ds(tb, _TPW)], idx)
        pltpu.async_copy(out_hbm.at[idx], buf, sem).wait()
        pltpu.sync_copy(buf, z1_hbm.at[pl.ds(tb, _TPW)])

    return dispatch, gather_back


def _swiglu_cached(xb, w1s, w2s, w3s):
    xb = xb.astype(jnp.bfloat16)
    h1 = lax.dot_general(xb, w1s[...], (((1,), (1,)), ((), ())),
                         preferred_element_type=jnp.float32)
    h3 = lax.dot_general(xb, w3s[...], (((1,), (1,)), ((), ())),
                         preferred_element_type=jnp.float32)
    h = (h1 / (1.0 + jnp.exp(-h1)) * h3).astype(jnp.bfloat16)
    return lax.dot_general(
        h, w2s[...], (((1,), (0,)), ((), ())),
        preferred_element_type=jnp.float32).astype(jnp.bfloat16)


def _ffn_body(be_ref, bv_ref, nx_ref, xg_ref, w1_hbm, w2_hbm, w3_hbm, o_ref,
              wfa, wfb, cache, sema, semb):
    b = pl.program_id(0)
    e = be_ref[b]
    bm1 = jnp.maximum(b - 1, 0)
    changed = jnp.logical_or(b == 0, e != be_ref[bm1])
    nxt = nx_ref[b]
    slot = e % 2
    nslot = nxt % 2

    def issue(ee, wf, sem):
        pltpu.async_copy(w1_hbm.at[ee], wf.at[0], sem.at[0])
        pltpu.async_copy(w2_hbm.at[ee], wf.at[1], sem.at[1])
        pltpu.async_copy(w3_hbm.at[ee], wf.at[2], sem.at[2])

    def drain(ee, wf, sem):
        pltpu.make_async_copy(w1_hbm.at[ee], wf.at[0], sem.at[0]).wait()
        pltpu.make_async_copy(w2_hbm.at[ee], wf.at[1], sem.at[1]).wait()
        pltpu.make_async_copy(w3_hbm.at[ee], wf.at[2], sem.at[2]).wait()

    for s, wf, sem in ((0, wfa, sema), (1, wfb, semb)):
        @pl.when(jnp.logical_and(b == 0, slot == s))
        def _(wf=wf, sem=sem):
            issue(e, wf, sem)

    @pl.when(changed)
    def _():
        for s, wf, sem in ((0, wfa, sema), (1, wfb, semb)):
            @pl.when(slot == s)
            def _(wf=wf, sem=sem):
                drain(e, wf, sem)
                cache[0, :, :] = wf[0].astype(jnp.bfloat16)
                cache[1, :, :] = wf[1].astype(jnp.bfloat16)
                cache[2, :, :] = wf[2].astype(jnp.bfloat16)

        for s, wf, sem in ((0, wfa, sema), (1, wfb, semb)):
            @pl.when(nslot == s)
            def _(wf=wf, sem=sem):
                issue(nxt, wf, sem)

    @pl.when(bv_ref[b] != 0)
    def _():
        xb = xg_ref[...].astype(jnp.bfloat16)
        h1 = lax.dot_general(xb, cache[0], (((1,), (1,)), ((), ())),
                             preferred_element_type=jnp.float32)
        h3 = lax.dot_general(xb, cache[2], (((1,), (1,)), ((), ())),
                             preferred_element_type=jnp.float32)
        h = (h1 / (1.0 + jnp.exp(-h1)) * h3).astype(jnp.bfloat16)
        o_ref[...] = lax.dot_general(h, cache[1], (((1,), (0,)), ((), ())),
                                     preferred_element_type=jnp.float32)

    for s, wf, sem in ((0, wfa, sema), (1, wfb, semb)):
        @pl.when(jnp.logical_and(b == _NB - 1, nslot == s))
        def _(wf=wf, sem=sem):
            drain(nxt, wf, sem)


_ffn_grid = pltpu.PrefetchScalarGridSpec(
    num_scalar_prefetch=3,
    grid=(_NB,),
    in_specs=[
        pl.BlockSpec((_BLK, _DIM), lambda b, be, bv, nx: (b, 0)),
        pl.BlockSpec(memory_space=pltpu.MemorySpace.HBM),
        pl.BlockSpec(memory_space=pltpu.MemorySpace.HBM),
        pl.BlockSpec(memory_space=pltpu.MemorySpace.HBM),
    ],
    out_specs=pl.BlockSpec((_BLK, _DIM), lambda b, be, bv, nx: (b, 0)),
    scratch_shapes=[
        pltpu.VMEM((3, _INTER, _DIM), jnp.float32),
        pltpu.VMEM((3, _INTER, _DIM), jnp.float32),
        pltpu.VMEM((3, _INTER, _DIM), jnp.bfloat16),
        pltpu.SemaphoreType.DMA((3,)),
        pltpu.SemaphoreType.DMA((3,)),
    ],
)

_ffn_call = pl.pallas_call(
    _ffn_body,
    grid_spec=_ffn_grid,
    out_shape=jax.ShapeDtypeStruct((_RPAD, _DIM), jnp.float32),
)


def _sffn_body(x_ref, w1_ref, w2_ref, w3_ref, o_ref, w1s, w2s, w3s):
    @pl.when(pl.program_id(0) == 0)
    def _():
        w1s[...] = w1_ref[0].astype(jnp.bfloat16)
        w2s[...] = w2_ref[0].astype(jnp.bfloat16)
        w3s[...] = w3_ref[0].astype(jnp.bfloat16)

    o_ref[...] = _swiglu_cached(x_ref[...], w1s, w2s, w3s)


_sffn_call = pl.pallas_call(
    _sffn_body,
    grid=(_T // _BLK,),
    in_specs=[
        pl.BlockSpec((_BLK, _DIM), lambda i: (i, 0)),
        pl.BlockSpec((1, _INTER, _DIM), lambda i: (0, 0, 0)),
        pl.BlockSpec((1, _INTER, _DIM), lambda i: (0, 0, 0)),
        pl.BlockSpec((1, _INTER, _DIM), lambda i: (0, 0, 0)),
    ],
    out_specs=pl.BlockSpec((_BLK, _DIM), lambda i: (i, 0)),
    out_shape=jax.ShapeDtypeStruct((_T, _DIM), jnp.bfloat16),
    scratch_shapes=[
        pltpu.VMEM((_INTER, _DIM), jnp.bfloat16),
        pltpu.VMEM((_INTER, _DIM), jnp.bfloat16),
        pltpu.VMEM((_INTER, _DIM), jnp.bfloat16),
    ],
)


_CB = 256


def _combine_body(z0_ref, z1_ref, os_ref, w0_ref, w1_ref, y_ref):
    w0 = w0_ref[:, 0:1]
    w1 = w1_ref[:, 0:1]
    y_ref[...] = (w0 * z0_ref[...].astype(jnp.float32)
                  + w1 * z1_ref[...].astype(jnp.float32)
                  + os_ref[...].astype(jnp.float32))


_combine_call = pl.pallas_call(
    _combine_body,
    grid=(_T // _CB,),
    in_specs=[
        pl.BlockSpec((_CB, _DIM), lambda i: (i, 0)),
        pl.BlockSpec((_CB, _DIM), lambda i: (i, 0)),
        pl.BlockSpec((_CB, _DIM), lambda i: (i, 0)),
        pl.BlockSpec((_CB, _E), lambda i: (i, 0)),
        pl.BlockSpec((_CB, _E), lambda i: (i, 0)),
    ],
    out_specs=pl.BlockSpec((_CB, _DIM), lambda i: (i, 0)),
    out_shape=jax.ShapeDtypeStruct((_T, _DIM), jnp.float32),
)


def kernel(x, gate_w, gate_bias, W1, W2, W3, SW1, SW2, SW3):
    gb = gate_bias.reshape(1, _E).astype(jnp.float32)
    dispatch, gather_back = _sc_kernels()
    dest, w0b, w1b, be, bv, nx = _gate_call(x, gate_w, gb)
    d0 = dest[0]
    d1 = dest[1]
    xg = dispatch(x, d0, d1)
    be_ = be.reshape(_NBPAD)[:_NB]
    bv_ = bv.reshape(_NBPAD)[:_NB]
    nx_ = nx.reshape(_NBPAD)[:_NB]
    out = _ffn_call(be_, bv_, nx_, xg, W1, jnp.swapaxes(W2, 1, 2), W3)
    z0, z1 = gather_back(out, d0, d1)
    out_s = _sffn_call(x, SW1, jnp.swapaxes(SW2, 1, 2), SW3)
    return _combine_call(z0, z1, out_s, w0b, w1b)


# speedup vs baseline: 1.9825x; 1.0004x over previous
import functools

import jax
import jax.numpy as jnp
from jax import lax
from jax.experimental import pallas as pl
from jax.experimental.pallas import tpu as pltpu
from jax.experimental.pallas import tpu_sc as plsc

_T = 2048
_DIM = 1024
_INTER = 704
_E = 8
_BLK = 256
_RPAD = _T * 2 + _E * _BLK
_L = _RPAD + _T
_NB = _RPAD // _BLK
_NBPAD = 64
_CHUNK = 256
_NCH = _T // _CHUNK
_NC = 2
_NS = 16
_NW = _NC * _NS
_TPW = _T // _NW
_DIMH = _DIM // 2


def _gate_body(x_ref, gw_ref, gb_ref, dest_ref, w0_ref, w1_ref, be_ref,
               bv_ref, nx_ref):
    x = x_ref[...]
    gw = gw_ref[...]
    logits = lax.dot_general(x, gw, (((1,), (1,)), ((), ())),
                             preferred_element_type=jnp.float32)
    scores = 1.0 / (1.0 + jnp.exp(-logits))
    biased = scores + gb_ref[...]
    e_iota = lax.broadcasted_iota(jnp.int32, (_T, _E), 1)

    m1 = jnp.max(biased, axis=1, keepdims=True)
    idx1 = jnp.min(jnp.where(biased >= m1, e_iota, _E), axis=1,
                   keepdims=True)
    oh0b = e_iota == idx1
    oh0 = oh0b.astype(jnp.float32)
    s1 = jnp.sum(oh0 * scores, axis=1, keepdims=True)
    biased2 = jnp.where(oh0b, -jnp.inf, biased)
    m2 = jnp.max(biased2, axis=1, keepdims=True)
    idx2 = jnp.min(jnp.where(biased2 >= m2, e_iota, _E), axis=1,
                   keepdims=True)
    oh1b = e_iota == idx2
    oh1 = oh1b.astype(jnp.float32)
    s2 = jnp.sum(oh1 * scores, axis=1, keepdims=True)
    denom = jnp.maximum(s1 + s2, 1e-10)
    w0_ref[...] = jnp.broadcast_to(s1 / denom, (_T, _E))
    w1_ref[...] = jnp.broadcast_to(s2 / denom, (_T, _E))

    r = lax.broadcasted_iota(jnp.int32, (_CHUNK, _CHUNK), 0)
    c = lax.broadcasted_iota(jnp.int32, (_CHUNK, _CHUNK), 1)
    tri = (r > c).astype(jnp.float32)
    r16 = lax.broadcasted_iota(jnp.int32, (_NCH, _NCH), 0)
    c16 = lax.broadcasted_iota(jnp.int32, (_NCH, _NCH), 1)
    tri16 = (r16 > c16).astype(jnp.float32)

    def excl_scan(oh):
        within, tots = [], []
        for k in range(_NCH):
            blk = oh[k * _CHUNK:(k + 1) * _CHUNK, :]
            within.append(jnp.dot(tri, blk, preferred_element_type=jnp.float32))
            tots.append(jnp.sum(blk, axis=0, keepdims=True))
        totals = jnp.concatenate(tots, axis=0)
        pref = jnp.dot(tri16, totals, preferred_element_type=jnp.float32)
        rows = [within[k] + pref[k:k + 1, :] for k in range(_NCH)]
        return jnp.concatenate(rows, axis=0), totals

    scan0, tot0_c = excl_scan(oh0)
    scan1, tot1_c = excl_scan(oh1)
    tot0 = jnp.sum(tot0_c, axis=0, keepdims=True)
    scan1 = scan1 + tot0
    cnt = tot0 + jnp.sum(tot1_c, axis=0, keepdims=True)

    cnt_i = cnt.astype(jnp.int32)
    pad_i = ((cnt_i + (_BLK - 1)) // _BLK) * _BLK
    pad = pad_i.astype(jnp.float32)
    r8 = lax.broadcasted_iota(jnp.int32, (_E, _E), 0)
    c8 = lax.broadcasted_iota(jnp.int32, (_E, _E), 1)
    up8 = (r8 < c8).astype(jnp.float32)
    offs = jnp.dot(pad, up8, preferred_element_type=jnp.float32)

    dest0 = jnp.sum(oh0 * (offs + scan0), axis=1)
    dest1 = jnp.sum(oh1 * (offs + scan1), axis=1)
    dest_ref[0, :] = dest0.astype(jnp.int32)
    dest_ref[1, :] = dest1.astype(jnp.int32)

    bstart = lax.broadcasted_iota(jnp.int32, (_NBPAD, 1), 0).astype(
        jnp.float32) * float(_BLK)
    offs9 = jnp.concatenate([offs, jnp.full((1, 1), float(_RPAD))], axis=1)
    cnt9 = jnp.concatenate([cnt, jnp.full((1, 1), float(_T))], axis=1)
    ge = (bstart >= offs9).astype(jnp.float32)
    blk_e = jnp.sum(ge, axis=1) - 1.0
    inseg = jnp.logical_and(bstart >= offs9, bstart < offs9 + cnt9)
    blk_v = jnp.sum(inseg.astype(jnp.float32), axis=1)
    be_ref[0, :] = blk_e.astype(jnp.int32)
    bv_ref[0, :] = blk_v.astype(jnp.int32)

    eidx = lax.broadcasted_iota(jnp.int32, (_NBPAD, _E), 1)
    blk_ei = blk_e.astype(jnp.int32)
    cand = jnp.logical_and(cnt > 0.0, eidx > blk_ei[:, None])
    nxt = jnp.min(jnp.where(cand, eidx, 99), axis=1)
    nx_ref[0, :] = jnp.where(nxt > 98, blk_ei, nxt)


_gate_call = pl.pallas_call(
    _gate_body,
    out_shape=[
        jax.ShapeDtypeStruct((2, _T), jnp.int32),
        jax.ShapeDtypeStruct((_T, _E), jnp.float32),
        jax.ShapeDtypeStruct((_T, _E), jnp.float32),
        jax.ShapeDtypeStruct((1, _NBPAD), jnp.int32),
        jax.ShapeDtypeStruct((1, _NBPAD), jnp.int32),
        jax.ShapeDtypeStruct((1, _NBPAD), jnp.int32),
    ],
)

@functools.cache
def _sc_kernels():
    mesh = plsc.VectorSubcoreMesh(core_axis_name="c", subcore_axis_name="s",
                                  num_cores=_NC, num_subcores=_NS)

    @functools.partial(
        pl.kernel,
        out_type=jax.ShapeDtypeStruct((_RPAD, _DIMH), jnp.int32),
        mesh=mesh,
        scratch_types=[
            pltpu.VMEM((_TPW, _DIMH), jnp.int32),
            pltpu.VMEM((_TPW,), jnp.int32),
            pltpu.VMEM((_TPW,), jnp.int32),
            pltpu.SemaphoreType.DMA,
            pltpu.SemaphoreType.DMA,
        ],
    )
    def dispatch(x_hbm, d0_hbm, d1_hbm, xg_hbm, xbuf, i0, i1, s0, s1):
        wid = lax.axis_index("s") * _NC + lax.axis_index("c")
        tb = wid * _TPW
        pltpu.sync_copy(x_hbm.at[pl.ds(tb, _TPW)], xbuf)
        pltpu.sync_copy(d0_hbm.at[pl.ds(tb, _TPW)], i0)
        pltpu.sync_copy(d1_hbm.at[pl.ds(tb, _TPW)], i1)
        c0 = pltpu.async_copy(xbuf, xg_hbm.at[i0], s0)
        c1 = pltpu.async_copy(xbuf, xg_hbm.at[i1], s1)
        c0.wait()
        c1.wait()

    @functools.partial(
        pl.kernel,
        out_type=(jax.ShapeDtypeStruct((_T, _DIM), jnp.float32),
                  jax.ShapeDtypeStruct((_T, _DIM), jnp.float32)),
        mesh=mesh,
        scratch_types=[
            pltpu.VMEM((_TPW, _DIM), jnp.float32),
            pltpu.VMEM((_TPW,), jnp.int32),
            pltpu.SemaphoreType.DMA,
        ],
    )
    def gather_back(out_hbm, d0_hbm, d1_hbm, z0_hbm, z1_hbm, buf, idx, sem):
        wid = lax.axis_index("s") * _NC + lax.axis_index("c")
        tb = wid * _TPW
        pltpu.sync_copy(d0_hbm.at[pl.ds(tb, _TPW)], idx)
        pltpu.async_copy(out_hbm.at[idx], buf, sem).wait()
        pltpu.sync_copy(buf, z0_hbm.at[pl.ds(tb, _TPW)])
        pltpu.sync_copy(d1_hbm.at[pl.ds(tb, _TPW)], idx)
        pltpu.async_copy(out_hbm.at[idx], buf, sem).wait()
        pltpu.sync_copy(buf, z1_hbm.at[pl.ds(tb, _TPW)])

    return dispatch, gather_back


def _swiglu_cached(xb, w1s, w2s, w3s):
    xb = xb.astype(jnp.bfloat16)
    h1 = lax.dot_general(xb, w1s[...], (((1,), (1,)), ((), ())),
                         preferred_element_type=jnp.float32)
    h3 = lax.dot_general(xb, w3s[...], (((1,), (1,)), ((), ())),
                         preferred_element_type=jnp.float32)
    h = (h1 / (1.0 + jnp.exp(-h1)) * h3).astype(jnp.bfloat16)
    return lax.dot_general(
        h, w2s[...], (((1,), (0,)), ((), ())),
        preferred_element_type=jnp.float32).astype(jnp.bfloat16)


def _ffn_body(be_ref, bv_ref, nx_ref, xg_ref, w1_hbm, w2_hbm, w3_hbm, o_ref,
              wfa, wfb, cache, sema, semb):
    b = pl.program_id(0)
    e = be_ref[b]
    bm1 = jnp.maximum(b - 1, 0)
    changed = jnp.logical_or(b == 0, e != be_ref[bm1])
    nxt = nx_ref[b]
    slot = e % 2
    nslot = nxt % 2

    def issue(ee, wf, sem):
        pltpu.async_copy(w1_hbm.at[ee], wf.at[0], sem.at[0])
        pltpu.async_copy(w2_hbm.at[ee], wf.at[1], sem.at[1])
        pltpu.async_copy(w3_hbm.at[ee], wf.at[2], sem.at[2])

    def drain(ee, wf, sem):
        pltpu.make_async_copy(w1_hbm.at[ee], wf.at[0], sem.at[0]).wait()
        pltpu.make_async_copy(w2_hbm.at[ee], wf.at[1], sem.at[1]).wait()
        pltpu.make_async_copy(w3_hbm.at[ee], wf.at[2], sem.at[2]).wait()

    for s, wf, sem in ((0, wfa, sema), (1, wfb, semb)):
        @pl.when(jnp.logical_and(b == 0, slot == s))
        def _(wf=wf, sem=sem):
            issue(e, wf, sem)

    @pl.when(changed)
    def _():
        for s, wf, sem in ((0, wfa, sema), (1, wfb, semb)):
            @pl.when(slot == s)
            def _(wf=wf, sem=sem):
                drain(e, wf, sem)
                cache[0, :, :] = wf[0].astype(jnp.bfloat16)
                cache[1, :, :] = wf[1].astype(jnp.bfloat16)
                cache[2, :, :] = wf[2].astype(jnp.bfloat16)

        for s, wf, sem in ((0, wfa, sema), (1, wfb, semb)):
            @pl.when(nslot == s)
            def _(wf=wf, sem=sem):
                issue(nxt, wf, sem)

    @pl.when(bv_ref[b] != 0)
    def _():
        xi = lax.bitcast_convert_type(xg_ref[...], jnp.uint32)
        lo = lax.bitcast_convert_type(xi << 16, jnp.float32)
        hi = lax.bitcast_convert_type(xi & jnp.uint32(0xFFFF0000), jnp.float32)
        xb = jnp.concatenate([lo, hi], axis=1).astype(jnp.bfloat16)
        h1 = lax.dot_general(xb, cache[0], (((1,), (1,)), ((), ())),
                             preferred_element_type=jnp.float32)
        h3 = lax.dot_general(xb, cache[2], (((1,), (1,)), ((), ())),
                             preferred_element_type=jnp.float32)
        h = (h1 / (1.0 + jnp.exp(-h1)) * h3).astype(jnp.bfloat16)
        o_ref[...] = lax.dot_general(h, cache[1], (((1,), (0,)), ((), ())),
                                     preferred_element_type=jnp.float32)

    for s, wf, sem in ((0, wfa, sema), (1, wfb, semb)):
        @pl.when(jnp.logical_and(b == _NB - 1, nslot == s))
        def _(wf=wf, sem=sem):
            drain(nxt, wf, sem)


_ffn_grid = pltpu.PrefetchScalarGridSpec(
    num_scalar_prefetch=3,
    grid=(_NB,),
    in_specs=[
        pl.BlockSpec((_BLK, _DIMH), lambda b, be, bv, nx: (b, 0)),
        pl.BlockSpec(memory_space=pltpu.MemorySpace.HBM),
        pl.BlockSpec(memory_space=pltpu.MemorySpace.HBM),
        pl.BlockSpec(memory_space=pltpu.MemorySpace.HBM),
    ],
    out_specs=pl.BlockSpec((_BLK, _DIM), lambda b, be, bv, nx: (b, 0)),
    scratch_shapes=[
        pltpu.VMEM((3, _INTER, _DIM), jnp.float32),
        pltpu.VMEM((3, _INTER, _DIM), jnp.float32),
        pltpu.VMEM((3, _INTER, _DIM), jnp.bfloat16),
        pltpu.SemaphoreType.DMA((3,)),
        pltpu.SemaphoreType.DMA((3,)),
    ],
)

_ffn_call = pl.pallas_call(
    _ffn_body,
    grid_spec=_ffn_grid,
    out_shape=jax.ShapeDtypeStruct((_RPAD, _DIM), jnp.float32),
)


def _sffn_body(x_ref, w1_ref, w2_ref, w3_ref, o_ref, w1s, w2s, w3s):
    @pl.when(pl.program_id(0) == 0)
    def _():
        w1s[...] = w1_ref[0].astype(jnp.bfloat16)
        w2s[...] = w2_ref[0].astype(jnp.bfloat16)
        w3s[...] = w3_ref[0].astype(jnp.bfloat16)

    o_ref[...] = _swiglu_cached(x_ref[...], w1s, w2s, w3s)


_sffn_call = pl.pallas_call(
    _sffn_body,
    grid=(_T // _BLK,),
    in_specs=[
        pl.BlockSpec((_BLK, _DIM), lambda i: (i, 0)),
        pl.BlockSpec((1, _INTER, _DIM), lambda i: (0, 0, 0)),
        pl.BlockSpec((1, _INTER, _DIM), lambda i: (0, 0, 0)),
        pl.BlockSpec((1, _INTER, _DIM), lambda i: (0, 0, 0)),
    ],
    out_specs=pl.BlockSpec((_BLK, _DIM), lambda i: (i, 0)),
    out_shape=jax.ShapeDtypeStruct((_T, _DIM), jnp.bfloat16),
    scratch_shapes=[
        pltpu.VMEM((_INTER, _DIM), jnp.bfloat16),
        pltpu.VMEM((_INTER, _DIM), jnp.bfloat16),
        pltpu.VMEM((_INTER, _DIM), jnp.bfloat16),
    ],
)


_CB = 256


def _combine_body(z0_ref, z1_ref, os_ref, w0_ref, w1_ref, y_ref):
    w0 = w0_ref[:, 0:1]
    w1 = w1_ref[:, 0:1]
    y_ref[...] = (w0 * z0_ref[...].astype(jnp.float32)
                  + w1 * z1_ref[...].astype(jnp.float32)
                  + os_ref[...].astype(jnp.float32))


_combine_call = pl.pallas_call(
    _combine_body,
    grid=(_T // _CB,),
    in_specs=[
        pl.BlockSpec((_CB, _DIM), lambda i: (i, 0)),
        pl.BlockSpec((_CB, _DIM), lambda i: (i, 0)),
        pl.BlockSpec((_CB, _DIM), lambda i: (i, 0)),
        pl.BlockSpec((_CB, _E), lambda i: (i, 0)),
        pl.BlockSpec((_CB, _E), lambda i: (i, 0)),
    ],
    out_specs=pl.BlockSpec((_CB, _DIM), lambda i: (i, 0)),
    out_shape=jax.ShapeDtypeStruct((_T, _DIM), jnp.float32),
)


def kernel(x, gate_w, gate_bias, W1, W2, W3, SW1, SW2, SW3):
    gb = gate_bias.reshape(1, _E).astype(jnp.float32)
    dispatch, gather_back = _sc_kernels()
    dest, w0b, w1b, be, bv, nx = _gate_call(x, gate_w, gb)
    d0 = dest[0]
    d1 = dest[1]
    xbf = x.astype(jnp.bfloat16)
    lo16 = lax.bitcast_convert_type(xbf[:, :_DIMH], jnp.uint16)
    hi16 = lax.bitcast_convert_type(xbf[:, _DIMH:], jnp.uint16)
    xpack = lax.bitcast_convert_type(
        lo16.astype(jnp.uint32) | (hi16.astype(jnp.uint32) << 16), jnp.int32)
    xg = dispatch(xpack, d0, d1)
    be_ = be.reshape(_NBPAD)[:_NB]
    bv_ = bv.reshape(_NBPAD)[:_NB]
    nx_ = nx.reshape(_NBPAD)[:_NB]
    out = _ffn_call(be_, bv_, nx_, xg, W1, jnp.swapaxes(W2, 1, 2), W3)
    z0, z1 = gather_back(out, d0, d1)
    out_s = _sffn_call(x, SW1, jnp.swapaxes(SW2, 1, 2), SW3)
    return _combine_call(z0, z1, out_s, w0b, w1b)


# speedup vs baseline: 1.9888x; 1.0032x over previous
import functools

import jax
import jax.numpy as jnp
from jax import lax
from jax.experimental import pallas as pl
from jax.experimental.pallas import tpu as pltpu
from jax.experimental.pallas import tpu_sc as plsc

_T = 2048
_DIM = 1024
_INTER = 704
_E = 8
_BLK = 256
_RPAD = _T * 2 + _E * _BLK
_L = _RPAD + _T
_NB = _RPAD // _BLK
_NBPAD = 64
_CHUNK = 256
_NCH = _T // _CHUNK
_NC = 2
_NS = 16
_NW = _NC * _NS
_TPW = _T // _NW


def _gate_body(x_ref, gw_ref, gb_ref, dest_ref, w0_ref, w1_ref, be_ref,
               bv_ref, nx_ref):
    x = x_ref[...]
    gw = gw_ref[...]
    logits = lax.dot_general(x, gw, (((1,), (1,)), ((), ())),
                             preferred_element_type=jnp.float32)
    scores = 1.0 / (1.0 + jnp.exp(-logits))
    biased = scores + gb_ref[...]
    e_iota = lax.broadcasted_iota(jnp.int32, (_T, _E), 1)

    m1 = jnp.max(biased, axis=1, keepdims=True)
    idx1 = jnp.min(jnp.where(biased >= m1, e_iota, _E), axis=1,
                   keepdims=True)
    oh0b = e_iota == idx1
    oh0 = oh0b.astype(jnp.float32)
    s1 = jnp.sum(oh0 * scores, axis=1, keepdims=True)
    biased2 = jnp.where(oh0b, -jnp.inf, biased)
    m2 = jnp.max(biased2, axis=1, keepdims=True)
    idx2 = jnp.min(jnp.where(biased2 >= m2, e_iota, _E), axis=1,
                   keepdims=True)
    oh1b = e_iota == idx2
    oh1 = oh1b.astype(jnp.float32)
    s2 = jnp.sum(oh1 * scores, axis=1, keepdims=True)
    denom = jnp.maximum(s1 + s2, 1e-10)
    w0_ref[...] = jnp.broadcast_to(s1 / denom, (_T, _E))
    w1_ref[...] = jnp.broadcast_to(s2 / denom, (_T, _E))

    r = lax.broadcasted_iota(jnp.int32, (_CHUNK, _CHUNK), 0)
    c = lax.broadcasted_iota(jnp.int32, (_CHUNK, _CHUNK), 1)
    tri = (r > c).astype(jnp.float32)
    r16 = lax.broadcasted_iota(jnp.int32, (_NCH, _NCH), 0)
    c16 = lax.broadcasted_iota(jnp.int32, (_NCH, _NCH), 1)
    tri16 = (r16 > c16).astype(jnp.float32)

    def excl_scan(oh):
        within, tots = [], []
        for k in range(_NCH):
            blk = oh[k * _CHUNK:(k + 1) * _CHUNK, :]
            within.append(jnp.dot(tri, blk, preferred_element_type=jnp.float32))
            tots.append(jnp.sum(blk, axis=0, keepdims=True))
        totals = jnp.concatenate(tots, axis=0)
        pref = jnp.dot(tri16, totals, preferred_element_type=jnp.float32)
        rows = [within[k] + pref[k:k + 1, :] for k in range(_NCH)]
        return jnp.concatenate(rows, axis=0), totals

    scan0, tot0_c = excl_scan(oh0)
    scan1, tot1_c = excl_scan(oh1)
    tot0 = jnp.sum(tot0_c, axis=0, keepdims=True)
    scan1 = scan1 + tot0
    cnt = tot0 + jnp.sum(tot1_c, axis=0, keepdims=True)

    cnt_i = cnt.astype(jnp.int32)
    pad_i = ((cnt_i + (_BLK - 1)) // _BLK) * _BLK
    pad = pad_i.astype(jnp.float32)
    r8 = lax.broadcasted_iota(jnp.int32, (_E, _E), 0)
    c8 = lax.broadcasted_iota(jnp.int32, (_E, _E), 1)
    up8 = (r8 < c8).astype(jnp.float32)
    offs = jnp.dot(pad, up8, preferred_element_type=jnp.float32)

    dest0 = jnp.sum(oh0 * (offs + scan0), axis=1)
    dest1 = jnp.sum(oh1 * (offs + scan1), axis=1)
    dest_ref[0, :] = dest0.astype(jnp.int32)
    dest_ref[1, :] = dest1.astype(jnp.int32)

    bstart = lax.broadcasted_iota(jnp.int32, (_NBPAD, 1), 0).astype(
        jnp.float32) * float(_BLK)
    offs9 = jnp.concatenate([offs, jnp.full((1, 1), float(_RPAD))], axis=1)
    cnt9 = jnp.concatenate([cnt, jnp.full((1, 1), float(_T))], axis=1)
    ge = (bstart >= offs9).astype(jnp.float32)
    blk_e = jnp.sum(ge, axis=1) - 1.0
    inseg = jnp.logical_and(bstart >= offs9, bstart < offs9 + cnt9)
    blk_v = jnp.sum(inseg.astype(jnp.float32), axis=1)
    be_ref[0, :] = blk_e.astype(jnp.int32)
    bv_ref[0, :] = blk_v.astype(jnp.int32)

    eidx = lax.broadcasted_iota(jnp.int32, (_NBPAD, _E), 1)
    blk_ei = blk_e.astype(jnp.int32)
    cand = jnp.logical_and(cnt > 0.0, eidx > blk_ei[:, None])
    nxt = jnp.min(jnp.where(cand, eidx, 99), axis=1)
    nx_ref[0, :] = jnp.where(nxt > 98, blk_ei, nxt)


_gate_call = pl.pallas_call(
    _gate_body,
    out_shape=[
        jax.ShapeDtypeStruct((2, _T), jnp.int32),
        jax.ShapeDtypeStruct((_T, _E), jnp.float32),
        jax.ShapeDtypeStruct((_T, _E), jnp.float32),
        jax.ShapeDtypeStruct((1, _NBPAD), jnp.int32),
        jax.ShapeDtypeStruct((1, _NBPAD), jnp.int32),
        jax.ShapeDtypeStruct((1, _NBPAD), jnp.int32),
    ],
)

@functools.cache
def _sc_kernels():
    mesh = plsc.VectorSubcoreMesh(core_axis_name="c", subcore_axis_name="s",
                                  num_cores=_NC, num_subcores=_NS)

    @functools.partial(
        pl.kernel,
        out_type=jax.ShapeDtypeStruct((_RPAD, _DIM), jnp.float32),
        mesh=mesh,
        scratch_types=[
            pltpu.VMEM((_TPW, _DIM), jnp.float32),
            pltpu.VMEM((_TPW,), jnp.int32),
            pltpu.VMEM((_TPW,), jnp.int32),
            pltpu.SemaphoreType.DMA,
            pltpu.SemaphoreType.DMA,
        ],
    )
    def dispatch(x_hbm, d0_hbm, d1_hbm, xg_hbm, xbuf, i0, i1, s0, s1):
        wid = lax.axis_index("s") * _NC + lax.axis_index("c")
        tb = wid * _TPW
        pltpu.sync_copy(x_hbm.at[pl.ds(tb, _TPW)], xbuf)
        pltpu.sync_copy(d0_hbm.at[pl.ds(tb, _TPW)], i0)
        pltpu.sync_copy(d1_hbm.at[pl.ds(tb, _TPW)], i1)
        c0 = pltpu.async_copy(xbuf, xg_hbm.at[i0], s0)
        c1 = pltpu.async_copy(xbuf, xg_hbm.at[i1], s1)
        c0.wait()
        c1.wait()

    @functools.partial(
        pl.kernel,
        out_type=(jax.ShapeDtypeStruct((_T, _DIM), jnp.float32),
                  jax.ShapeDtypeStruct((_T, _DIM), jnp.float32)),
        mesh=mesh,
        scratch_types=[
            pltpu.VMEM((_TPW, _DIM), jnp.float32),
            pltpu.VMEM((_TPW,), jnp.int32),
            pltpu.SemaphoreType.DMA,
        ],
    )
    def gather_back(out_hbm, d0_hbm, d1_hbm, z0_hbm, z1_hbm, buf, idx, sem):
        wid = lax.axis_index("s") * _NC + lax.axis_index("c")
        tb = wid * _TPW
        pltpu.sync_copy(d0_hbm.at[pl.ds(tb, _TPW)], idx)
        pltpu.async_copy(out_hbm.at[idx], buf, sem).wait()
        pltpu.sync_copy(buf, z0_hbm.at[pl.ds(tb, _TPW)])
        pltpu.sync_copy(d1_hbm.at[pl.ds(tb, _TPW)], idx)
        pltpu.async_copy(out_hbm.at[idx], buf, sem).wait()
        pltpu.sync_copy(buf, z1_hbm.at[pl.ds(tb, _TPW)])

    return dispatch, gather_back


def _swiglu_cached(xb, w1s, w2s, w3s):
    xb = xb.astype(jnp.bfloat16)
    h1 = lax.dot_general(xb, w1s[...], (((1,), (1,)), ((), ())),
                         preferred_element_type=jnp.float32)
    h3 = lax.dot_general(xb, w3s[...], (((1,), (1,)), ((), ())),
                         preferred_element_type=jnp.float32)
    h = (h1 / (1.0 + jnp.exp(-h1)) * h3).astype(jnp.bfloat16)
    return lax.dot_general(
        h, w2s[...], (((1,), (0,)), ((), ())),
        preferred_element_type=jnp.float32).astype(jnp.bfloat16)


def _ffn_body(be_ref, bv_ref, nx_ref, xg_ref, w1_hbm, w2_hbm, w3_hbm, o_ref,
              wfa, wfb, cache, sema, semb):
    b = pl.program_id(0)
    e = be_ref[b]
    bm1 = jnp.maximum(b - 1, 0)
    changed = jnp.logical_or(b == 0, e != be_ref[bm1])
    nxt = nx_ref[b]
    slot = e % 2
    nslot = nxt % 2

    def issue(ee, wf, sem):
        pltpu.async_copy(w1_hbm.at[ee], wf.at[0], sem.at[0])
        pltpu.async_copy(w2_hbm.at[ee], wf.at[1], sem.at[1])
        pltpu.async_copy(w3_hbm.at[ee], wf.at[2], sem.at[2])

    def drain(ee, wf, sem):
        pltpu.make_async_copy(w1_hbm.at[ee], wf.at[0], sem.at[0]).wait()
        pltpu.make_async_copy(w2_hbm.at[ee], wf.at[1], sem.at[1]).wait()
        pltpu.make_async_copy(w3_hbm.at[ee], wf.at[2], sem.at[2]).wait()

    for s, wf, sem in ((0, wfa, sema), (1, wfb, semb)):
        @pl.when(jnp.logical_and(b == 0, slot == s))
        def _(wf=wf, sem=sem):
            issue(e, wf, sem)

    @pl.when(changed)
    def _():
        for s, wf, sem in ((0, wfa, sema), (1, wfb, semb)):
            @pl.when(slot == s)
            def _(wf=wf, sem=sem):
                drain(e, wf, sem)
                cache[0, :, :] = wf[0].astype(jnp.bfloat16)
                cache[1, :, :] = wf[1].astype(jnp.bfloat16)
                cache[2, :, :] = wf[2].astype(jnp.bfloat16)

        for s, wf, sem in ((0, wfa, sema), (1, wfb, semb)):
            @pl.when(nslot == s)
            def _(wf=wf, sem=sem):
                issue(nxt, wf, sem)

    @pl.when(bv_ref[b] != 0)
    def _():
        xb = xg_ref[...].astype(jnp.bfloat16)
        h1 = lax.dot_general(xb, cache[0], (((1,), (1,)), ((), ())),
                             preferred_element_type=jnp.float32)
        h3 = lax.dot_general(xb, cache[2], (((1,), (1,)), ((), ())),
                             preferred_element_type=jnp.float32)
        h = (h1 / (1.0 + jnp.exp(-h1)) * h3).astype(jnp.bfloat16)
        o_ref[...] = lax.dot_general(h, cache[1], (((1,), (0,)), ((), ())),
                                     preferred_element_type=jnp.float32)

    for s, wf, sem in ((0, wfa, sema), (1, wfb, semb)):
        @pl.when(jnp.logical_and(b == _NB - 1, nslot == s))
        def _(wf=wf, sem=sem):
            drain(nxt, wf, sem)


_ffn_grid = pltpu.PrefetchScalarGridSpec(
    num_scalar_prefetch=3,
    grid=(_NB,),
    in_specs=[
        pl.BlockSpec((_BLK, _DIM), lambda b, be, bv, nx: (b, 0)),
        pl.BlockSpec(memory_space=pltpu.MemorySpace.HBM),
        pl.BlockSpec(memory_space=pltpu.MemorySpace.HBM),
        pl.BlockSpec(memory_space=pltpu.MemorySpace.HBM),
    ],
    out_specs=pl.BlockSpec((_BLK, _DIM), lambda b, be, bv, nx: (b, 0)),
    scratch_shapes=[
        pltpu.VMEM((3, _INTER, _DIM), jnp.float32),
        pltpu.VMEM((3, _INTER, _DIM), jnp.float32),
        pltpu.VMEM((3, _INTER, _DIM), jnp.bfloat16),
        pltpu.SemaphoreType.DMA((3,)),
        pltpu.SemaphoreType.DMA((3,)),
    ],
)

_ffn_call = pl.pallas_call(
    _ffn_body,
    grid_spec=_ffn_grid,
    out_shape=jax.ShapeDtypeStruct((_RPAD, _DIM), jnp.float32),
)


def _sffn_body(x_ref, w1_ref, w2_ref, w3_ref, o_ref, w1s, w2s, w3s):
    @pl.when(pl.program_id(0) == 0)
    def _():
        w1s[...] = w1_ref[0].astype(jnp.bfloat16)
        w2s[...] = w2_ref[0].astype(jnp.bfloat16)
        w3s[...] = w3_ref[0].astype(jnp.bfloat16)

    o_ref[...] = _swiglu_cached(x_ref[...], w1s, w2s, w3s)


_sffn_call = pl.pallas_call(
    _sffn_body,
    grid=(_T // _BLK,),
    in_specs=[
        pl.BlockSpec((_BLK, _DIM), lambda i: (i, 0)),
        pl.BlockSpec((1, _INTER, _DIM), lambda i: (0, 0, 0)),
        pl.BlockSpec((1, _INTER, _DIM), lambda i: (0, 0, 0)),
        pl.BlockSpec((1, _INTER, _DIM), lambda i: (0, 0, 0)),
    ],
    out_specs=pl.BlockSpec((_BLK, _DIM), lambda i: (i, 0)),
    out_shape=jax.ShapeDtypeStruct((_T, _DIM), jnp.bfloat16),
    scratch_shapes=[
        pltpu.VMEM((_INTER, _DIM), jnp.bfloat16),
        pltpu.VMEM((_INTER, _DIM), jnp.bfloat16),
        pltpu.VMEM((_INTER, _DIM), jnp.bfloat16),
    ],
)


_CB = 256


def _combine_body(z0_ref, z1_ref, os_ref, w0_ref, w1_ref, y_ref):
    w0 = w0_ref[:, 0:1]
    w1 = w1_ref[:, 0:1]
    y_ref[...] = (w0 * z0_ref[...].astype(jnp.float32)
                  + w1 * z1_ref[...].astype(jnp.float32)
                  + os_ref[...].astype(jnp.float32))


_combine_call = pl.pallas_call(
    _combine_body,
    grid=(_T // _CB,),
    in_specs=[
        pl.BlockSpec((_CB, _DIM), lambda i: (i, 0)),
        pl.BlockSpec((_CB, _DIM), lambda i: (i, 0)),
        pl.BlockSpec((_CB, _DIM), lambda i: (i, 0)),
        pl.BlockSpec((_CB, _E), lambda i: (i, 0)),
        pl.BlockSpec((_CB, _E), lambda i: (i, 0)),
    ],
    out_specs=pl.BlockSpec((_CB, _DIM), lambda i: (i, 0)),
    out_shape=jax.ShapeDtypeStruct((_T, _DIM), jnp.float32),
)


def kernel(x, gate_w, gate_bias, W1, W2, W3, SW1, SW2, SW3):
    gb = gate_bias.reshape(1, _E).astype(jnp.float32)
    dispatch, gather_back = _sc_kernels()
    dest, w0b, w1b, be, bv, nx = _gate_call(x, gate_w, gb)
    d0 = dest[0]
    d1 = dest[1]
    xg = dispatch(x, d0, d1)
    be_ = be.reshape(_NBPAD)[:_NB]
    bv_ = bv.reshape(_NBPAD)[:_NB]
    nx_ = nx.reshape(_NBPAD)[:_NB]
    out = _ffn_call(be_, bv_, nx_, xg, W1, jnp.swapaxes(W2, 1, 2), W3)
    z0, z1 = gather_back(out, d0, d1)
    out_s = _sffn_call(x, SW1, jnp.swapaxes(SW2, 1, 2), SW3)
    return _combine_call(z0, z1, out_s, w0b, w1b)
